# baseline (speedup 1.0000x reference)
import functools

import jax
import jax.numpy as jnp
import numpy as np
from jax.experimental import pallas as pl

B = 8
N = 4096
F_IN = 3
NUM_CLASSES = 13
RATIOS = (0.2, 0.25)
RADII = (0.2, 0.4)
MAX_NEIGH = 64


def _apply_mlp(ps, x):
    n = len(ps)
    for i, (W, b) in enumerate(ps):
        x = x @ W + b
        if i < n - 1:
            x = jax.nn.relu(x)
    return x


def _fps_single(pos, n_sample):
    dists0 = jnp.full((pos.shape[0],), jnp.inf, jnp.float32)

    def step(carry, _):
        last, dists = carry
        d = jnp.sum((pos - pos[last]) ** 2, axis=-1)
        dists = jnp.minimum(dists, d)
        nxt = jnp.argmax(dists).astype(jnp.int32)
        return (nxt, dists), nxt

    (_, _), rest = jax.lax.scan(step, (jnp.int32(0), dists0), None, length=n_sample - 1)
    return jnp.concatenate([jnp.zeros((1,), jnp.int32), rest])


def _radius_knn(pos_all, pos_q, r, k):
    d2 = jnp.sum((pos_q[:, None, :] - pos_all[None, :, :]) ** 2, axis=-1)
    d2m = jnp.where(d2 <= r * r, d2, jnp.inf)
    neg, idx = jax.lax.top_k(-d2m, k)
    return idx, jnp.isfinite(neg)


def _sa_module(ps, x, pos, ratio, r):
    S = max(1, int(round(pos.shape[1] * ratio)))
    idx = jax.vmap(lambda p: _fps_single(p, S))(pos)
    pos_dst = jnp.take_along_axis(pos, idx[..., None], axis=1)
    nidx, valid = jax.vmap(lambda pa, pq: _radius_knn(pa, pq, r, MAX_NEIGH))(pos, pos_dst)
    pos_j = jax.vmap(lambda p, i: p[i])(pos, nidx)
    x_j = jax.vmap(lambda xs, i: xs[i])(x, nidx)
    rel = pos_j - pos_dst[:, :, None, :]
    msg = _apply_mlp(ps, jnp.concatenate([x_j, rel], axis=-1))
    msg = jnp.where(valid[..., None], msg, -jnp.inf)
    return jnp.max(msg, axis=2), pos_dst


def _global_sa(ps, x, pos):
    h = _apply_mlp(ps, jnp.concatenate([x, pos], axis=-1))
    xg = jnp.max(h, axis=1, keepdims=True)
    posg = jnp.zeros((x.shape[0], 1, 3), jnp.float32)
    return xg, posg


def _knn_interpolate(x, pos, pos_skip, k):
    d2 = jnp.sum((pos_skip[:, :, None, :] - pos[:, None, :, :]) ** 2, axis=-1)
    neg, idx = jax.lax.top_k(-d2, k)
    w = 1.0 / jnp.maximum(-neg, 1e-16)
    w = w / jnp.sum(w, axis=-1, keepdims=True)
    xk = jax.vmap(lambda xs, i: xs[i])(x, idx)
    return jnp.sum(w[..., None] * xk, axis=2)


def _fp_module(ps, k, x, pos, x_skip, pos_skip):
    xi = _knn_interpolate(x, pos, pos_skip, k)
    xi = jnp.concatenate([xi, x_skip], axis=-1)
    return _apply_mlp(ps, xi)


def _head_kernel(f_ref, w0, b0, w1, b1, w2, b2, o_ref):
    h = f_ref[...]
    h = jnp.maximum(h @ w0[...] + b0[...][None, :], 0.0)
    h = jnp.maximum(h @ w1[...] + b1[...][None, :], 0.0)
    o = h @ w2[...] + b2[...][None, :]
    o = o - jax.scipy.special.logsumexp(o, axis=-1, keepdims=True)
    o_ref[...] = o


def _head(f1, ps):
    (w0, b0), (w1, b1), (w2, b2) = ps
    M = f1.shape[0]
    TILE = 2048
    grid = (M // TILE,)
    return pl.pallas_call(
        _head_kernel,
        grid=grid,
        in_specs=[
            pl.BlockSpec((TILE, f1.shape[1]), lambda i: (i, 0)),
            pl.BlockSpec(w0.shape, lambda i: (0, 0)),
            pl.BlockSpec(b0.shape, lambda i: (0,)),
            pl.BlockSpec(w1.shape, lambda i: (0, 0)),
            pl.BlockSpec(b1.shape, lambda i: (0,)),
            pl.BlockSpec(w2.shape, lambda i: (0, 0)),
            pl.BlockSpec(b2.shape, lambda i: (0,)),
        ],
        out_specs=pl.BlockSpec((TILE, NUM_CLASSES), lambda i: (i, 0)),
        out_shape=jax.ShapeDtypeStruct((M, NUM_CLASSES), jnp.float32),
    )(f1, w0, b0, w1, b1, w2, b2)


def kernel(x, pos, batch, params):
    del batch
    x0 = x.reshape(B, N, F_IN)
    p0 = pos.reshape(B, N, 3)
    x1, p1 = _sa_module(params['sa1'], x0, p0, RATIOS[0], RADII[0])
    x2, p2 = _sa_module(params['sa2'], x1, p1, RATIOS[1], RADII[1])
    x3, p3 = _global_sa(params['sa3'], x2, p2)
    f3 = _fp_module(params['fp3'], 1, x3, p3, x2, p2)
    f2 = _fp_module(params['fp2'], 3, f3, p2, x1, p1)
    f1 = _fp_module(params['fp1'], 3, f2, p1, x0, p0)
    out = _head(f1.reshape(B * N, -1), params['out'])
    return out.reshape(B * N, NUM_CLASSES)


# speedup vs baseline: 3.7276x; 3.7276x over previous
import functools

import jax
import jax.numpy as jnp
import numpy as np
from jax.experimental import pallas as pl

B = 8
N = 4096
F_IN = 3
NUM_CLASSES = 13
RATIOS = (0.2, 0.25)
RADII = (0.2, 0.4)
MAX_NEIGH = 64


S1, S1P = 819, 832
S2, S2P = 205, 256
NP1 = 896
PADPOS = 1e9


def _tile_store(ref, t, val_col):
    Bb = val_col.shape[0]
    tbase = pl.multiple_of((t // 128) * 128, 128)
    lane = jax.lax.broadcasted_iota(jnp.int32, (Bb, 128), 1)
    sel = lane == (t % 128)
    cur = ref[:, pl.ds(tbase, 128)]
    ref[:, pl.ds(tbase, 128)] = jnp.where(sel, jnp.broadcast_to(val_col, (Bb, 128)), cur)


def _fps_kernel(px_ref, py_ref, pz_ref, d0_ref, idx_ref, sx_ref, sy_ref, sz_ref,
                *, n_samples):
    posx = px_ref[...]
    posy = py_ref[...]
    posz = pz_ref[...]
    np_lanes = posx.shape[1]
    Bb = posx.shape[0]
    lane = jax.lax.broadcasted_iota(jnp.int32, posx.shape, 1)
    idx_ref[...] = jnp.zeros(idx_ref.shape, jnp.int32)
    sx_ref[...] = jnp.zeros(sx_ref.shape, jnp.float32)
    sy_ref[...] = jnp.zeros(sy_ref.shape, jnp.float32)
    sz_ref[...] = jnp.zeros(sz_ref.shape, jnp.float32)

    def step(t, carry):
        last, dists = carry
        onehot = lane == last
        px = jnp.sum(jnp.where(onehot, posx, 0.0), axis=1, keepdims=True)
        py = jnp.sum(jnp.where(onehot, posy, 0.0), axis=1, keepdims=True)
        pz = jnp.sum(jnp.where(onehot, posz, 0.0), axis=1, keepdims=True)
        dx = posx - px
        dy = posy - py
        dz = posz - pz
        d2 = (dx * dx + dy * dy) + dz * dz
        dists = jnp.minimum(dists, d2)
        m = jnp.max(dists, axis=1, keepdims=True)
        cand = jnp.where(dists == m, lane, np_lanes)
        nxt = jnp.min(cand, axis=1, keepdims=True)
        _tile_store(idx_ref, t, nxt)
        _tile_store(sx_ref, t - 1, px)
        _tile_store(sy_ref, t - 1, py)
        _tile_store(sz_ref, t - 1, pz)
        return nxt, dists

    last, _ = jax.lax.fori_loop(
        1, n_samples, step,
        (jnp.zeros((Bb, 1), jnp.int32), d0_ref[...]))
    onehot = lane == last
    _tile_store(sx_ref, n_samples - 1,
                jnp.sum(jnp.where(onehot, posx, 0.0), axis=1, keepdims=True))
    _tile_store(sy_ref, n_samples - 1,
                jnp.sum(jnp.where(onehot, posy, 0.0), axis=1, keepdims=True))
    _tile_store(sz_ref, n_samples - 1,
                jnp.sum(jnp.where(onehot, posz, 0.0), axis=1, keepdims=True))


def _fps(posx, posy, posz, n_real, n_samples):
    Bb, Np = posx.shape
    sp = -n_samples % 128 + n_samples
    lane = jax.lax.broadcasted_iota(jnp.int32, (Bb, Np), 1)
    d0 = jnp.where(lane < n_real, jnp.inf, -jnp.inf).astype(jnp.float32)
    out_shapes = (
        jax.ShapeDtypeStruct((Bb, sp), jnp.int32),
        jax.ShapeDtypeStruct((Bb, sp), jnp.float32),
        jax.ShapeDtypeStruct((Bb, sp), jnp.float32),
        jax.ShapeDtypeStruct((Bb, sp), jnp.float32),
    )
    idx, sx, sy, sz = pl.pallas_call(
        functools.partial(_fps_kernel, n_samples=n_samples),
        out_shape=out_shapes,
    )(posx, posy, posz, d0)
    return (idx[:, :n_samples], sx[:, :n_samples], sy[:, :n_samples],
            sz[:, :n_samples])


_SENT = 0x7F000000


def _select_kernel(qx_ref, qy_ref, qz_ref, px_ref, py_ref, pz_ref,
                   keys_ref, tau_ref, *, r2, k):
    qx = qx_ref[0]
    qy = qy_ref[0]
    qz = qz_ref[0]
    px = px_ref[0]
    py = py_ref[0]
    pz = pz_ref[0]
    dx = qx - px
    dy = qy - py
    dz = qz - pz
    d2 = (dx * dx + dy * dy) + dz * dz
    lane = jax.lax.broadcasted_iota(jnp.int32, d2.shape, 1)
    bits = jax.lax.bitcast_convert_type(d2, jnp.int32)
    keys = jnp.where(d2 <= r2, (bits & (~0xFFF)) | lane, _SENT)
    keys_ref[0] = keys

    q = d2.shape[0]
    lo = jnp.full((q, 1), -1, jnp.int32)
    hi = jnp.full((q, 1), _SENT, jnp.int32)

    def body(_, carry):
        lo, hi = carry
        mid = lo + (hi - lo) // 2
        cnt = jnp.sum((keys <= mid).astype(jnp.int32), axis=1, keepdims=True)
        pred = cnt >= k
        return jnp.where(pred, lo, mid), jnp.where(pred, mid, hi)

    lo, hi = jax.lax.fori_loop(0, 31, body, (lo, hi))
    tau_ref[0] = jnp.minimum(hi, _SENT - 1)


def _select(qx, qy, qz, px, py, pz, r2, k, qblk):
    Bb, Qp = qx.shape
    Np = px.shape[1]
    q3 = qx.reshape(Bb, Qp, 1)
    grid = (Bb, Qp // qblk)
    qspec = pl.BlockSpec((1, qblk, 1), lambda b, i: (b, i, 0))
    pspec = pl.BlockSpec((1, 1, Np), lambda b, i: (b, 0, 0))
    return pl.pallas_call(
        functools.partial(_select_kernel, r2=r2, k=k),
        grid=grid,
        in_specs=[qspec, qspec, qspec, pspec, pspec, pspec],
        out_specs=[
            pl.BlockSpec((1, qblk, Np), lambda b, i: (b, i, 0)),
            pl.BlockSpec((1, qblk, 1), lambda b, i: (b, i, 0)),
        ],
        out_shape=[
            jax.ShapeDtypeStruct((Bb, Qp, Np), jnp.int32),
            jax.ShapeDtypeStruct((Bb, Qp, 1), jnp.int32),
        ],
    )(qx.reshape(Bb, Qp, 1), qy.reshape(Bb, Qp, 1), qz.reshape(Bb, Qp, 1),
      px.reshape(Bb, 1, Np), py.reshape(Bb, 1, Np), pz.reshape(Bb, 1, Np))


def _sa_mlp_kernel(g_ref, qpr_ref, cntr_ref, cnt_ref, w1_ref, w1p_ref, b1_ref,
                   w2_ref, b2_ref, w3_ref, b3_ref, o_ref, *, nneigh):
    g = g_ref[...]
    rows = g.shape[0]
    qb = rows // nneigh
    tq = jnp.dot(qpr_ref[...], w1p_ref[...], preferred_element_type=jnp.float32)
    h = jnp.dot(g, w1_ref[...], preferred_element_type=jnp.float32)
    h = jnp.maximum(h + b1_ref[...][None, :] - tq, 0.0)
    h = jnp.maximum(jnp.dot(h, w2_ref[...], preferred_element_type=jnp.float32)
                    + b2_ref[...][None, :], 0.0)
    h = jnp.dot(h, w3_ref[...], preferred_element_type=jnp.float32) + b3_ref[...][None, :]
    slot = jax.lax.broadcasted_iota(jnp.int32, (rows, 1), 0) % nneigh
    h = jnp.where(slot < cntr_ref[...], h, -jnp.inf)
    m = jnp.max(h.reshape(qb, nneigh, h.shape[-1]), axis=1)
    o_ref[...] = jnp.where(cnt_ref[...] > 0, m, 0.0)


def _sa_mlp(G, qpos, counts, ps, nneigh, cin_split, qblk):
    (W1, b1), (W2, b2), (W3, b3) = ps
    Q, Cpad = G.shape[0] // nneigh, G.shape[1]
    W1x = W1[:cin_split]
    W1p = W1[cin_split:cin_split + 3]
    W1pad = jnp.zeros((Cpad, W1.shape[1]), jnp.float32)
    W1pad = W1pad.at[:cin_split].set(W1x).at[cin_split:cin_split + 3].set(W1p)
    qpos_rep = jnp.repeat(qpos, nneigh, axis=0)
    cnt_rep = jnp.repeat(counts, nneigh, axis=0)
    grid = (Q // qblk,)
    return pl.pallas_call(
        functools.partial(_sa_mlp_kernel, nneigh=nneigh),
        grid=grid,
        in_specs=[
            pl.BlockSpec((qblk * nneigh, Cpad), lambda i: (i, 0)),
            pl.BlockSpec((qblk * nneigh, 3), lambda i: (i, 0)),
            pl.BlockSpec((qblk * nneigh, 1), lambda i: (i, 0)),
            pl.BlockSpec((qblk, 1), lambda i: (i, 0)),
            pl.BlockSpec(W1pad.shape, lambda i: (0, 0)),
            pl.BlockSpec(W1p.shape, lambda i: (0, 0)),
            pl.BlockSpec(b1.shape, lambda i: (0,)),
            pl.BlockSpec(W2.shape, lambda i: (0, 0)),
            pl.BlockSpec(b2.shape, lambda i: (0,)),
            pl.BlockSpec(W3.shape, lambda i: (0, 0)),
            pl.BlockSpec(b3.shape, lambda i: (0,)),
        ],
        out_specs=pl.BlockSpec((qblk, W3.shape[1]), lambda i: (i, 0)),
        out_shape=jax.ShapeDtypeStruct((Q, W3.shape[1]), jnp.float32),
    )(G, qpos_rep, cnt_rep, counts, W1pad, W1p, b1, W2, b2, W3, b3)


def _sa3_fp3_kernel(x2_ref, qp_ref, w1_ref, w1p_ref, b1_ref, w2_ref, b2_ref,
                    w3_ref, b3_ref, f1a_ref, f1b_ref, fb1_ref, f2w_ref,
                    fb2_ref, o_ref, *, n_real):
    x2 = x2_ref[0]
    qp = qp_ref[0]
    h = jnp.dot(x2, w1_ref[...], preferred_element_type=jnp.float32)
    h = h + jnp.dot(qp, w1p_ref[...], preferred_element_type=jnp.float32)
    h = jnp.maximum(h + b1_ref[...][None, :], 0.0)
    h = jnp.maximum(jnp.dot(h, w2_ref[...], preferred_element_type=jnp.float32)
                    + b2_ref[...][None, :], 0.0)
    h = jnp.dot(h, w3_ref[...], preferred_element_type=jnp.float32) + b3_ref[...][None, :]
    row = jax.lax.broadcasted_iota(jnp.int32, h.shape, 0)
    h = jnp.where(row < n_real, h, -jnp.inf)
    xg = jnp.max(h, axis=0, keepdims=True)
    f = jnp.dot(x2, f1b_ref[...], preferred_element_type=jnp.float32)
    f = f + jnp.dot(xg, f1a_ref[...], preferred_element_type=jnp.float32)
    f = jnp.maximum(f + fb1_ref[...][None, :], 0.0)
    f = jnp.dot(f, f2w_ref[...], preferred_element_type=jnp.float32) + fb2_ref[...][None, :]
    row2 = jax.lax.broadcasted_iota(jnp.int32, f.shape, 0)
    o_ref[0] = jnp.where(row2 < n_real, f, 0.0)


def _sa3_fp3(x2, pos2, ps3, psf, n_real):
    (W1, b1), (W2, b2), (W3, b3) = ps3
    (F1, fb1), (F2, fb2) = psf
    Bb, S2p, C = x2.shape
    W1x = W1[:C]
    W1p = W1[C:C + 3]
    F1a = F1[:W3.shape[1]]
    F1b = F1[W3.shape[1]:]
    grid = (Bb,)
    return pl.pallas_call(
        functools.partial(_sa3_fp3_kernel, n_real=n_real),
        grid=grid,
        in_specs=[
            pl.BlockSpec((1, S2p, C), lambda b: (b, 0, 0)),
            pl.BlockSpec((1, S2p, 3), lambda b: (b, 0, 0)),
            pl.BlockSpec(W1x.shape, lambda b: (0, 0)),
            pl.BlockSpec(W1p.shape, lambda b: (0, 0)),
            pl.BlockSpec(b1.shape, lambda b: (0,)),
            pl.BlockSpec(W2.shape, lambda b: (0, 0)),
            pl.BlockSpec(b2.shape, lambda b: (0,)),
            pl.BlockSpec(W3.shape, lambda b: (0, 0)),
            pl.BlockSpec(b3.shape, lambda b: (0,)),
            pl.BlockSpec(F1a.shape, lambda b: (0, 0)),
            pl.BlockSpec(F1b.shape, lambda b: (0, 0)),
            pl.BlockSpec(fb1.shape, lambda b: (0,)),
            pl.BlockSpec(F2.shape, lambda b: (0, 0)),
            pl.BlockSpec(fb2.shape, lambda b: (0,)),
        ],
        out_specs=pl.BlockSpec((1, S2p, F2.shape[1]), lambda b: (b, 0, 0)),
        out_shape=jax.ShapeDtypeStruct((Bb, S2p, F2.shape[1]), jnp.float32),
    )(x2, pos2, W1x, W1p, b1, W2, b2, W3, b3, F1a, F1b, fb1, F2, fb2)


def _interp_kernel(qx_ref, qy_ref, qz_ref, px_ref, py_ref, pz_ref, f_ref,
                   xskip_ref, wa_ref, wb_ref, b1_ref, *rest, n_layers):
    if n_layers == 2:
        w2_ref, b2_ref, o_ref = rest
    else:
        w2_ref, b2_ref, w3_ref, b3_ref, o_ref = rest
    qx = qx_ref[0]
    qy = qy_ref[0]
    qz = qz_ref[0]
    px = px_ref[0]
    py = py_ref[0]
    pz = pz_ref[0]
    dx = qx - px
    dy = qy - py
    dz = qz - pz
    d2 = (dx * dx + dy * dy) + dz * dz
    lane = jax.lax.broadcasted_iota(jnp.int32, d2.shape, 1)
    qb, ncand = d2.shape
    Wint = jnp.zeros(d2.shape, jnp.float32)
    wsum = jnp.zeros((qb, 1), jnp.float32)
    for _ in range(3):
        m = jnp.min(d2, axis=1, keepdims=True)
        cand = jnp.where(d2 == m, lane, ncand)
        idxk = jnp.min(cand, axis=1, keepdims=True)
        w = 1.0 / jnp.maximum(m, 1e-16)
        hit = lane == idxk
        Wint = jnp.where(hit, w, Wint)
        wsum = wsum + w
        d2 = jnp.where(hit, jnp.inf, d2)
    Wint = Wint / wsum
    xi = jnp.dot(Wint, f_ref[0], preferred_element_type=jnp.float32)
    h = jnp.dot(xi, wa_ref[...], preferred_element_type=jnp.float32)
    h = h + jnp.dot(xskip_ref[0], wb_ref[...], preferred_element_type=jnp.float32)
    h = jnp.maximum(h + b1_ref[...][None, :], 0.0)
    h = jnp.dot(h, w2_ref[...], preferred_element_type=jnp.float32) + b2_ref[...][None, :]
    if n_layers == 3:
        h = jnp.maximum(h, 0.0)
        h = jnp.dot(h, w3_ref[...], preferred_element_type=jnp.float32) + b3_ref[...][None, :]
    o_ref[0] = h


def _interp_mlp(qplanes, cplanes, f_src, x_skip, ps, c_src, qblk):
    (W1, b1), *restps = ps
    Bb, Qp = qplanes[0].shape
    Ncp = cplanes[0].shape[1]
    Cs = x_skip.shape[-1]
    Wa = W1[:c_src]
    Wb = W1[c_src:]
    n_layers = 1 + len(restps)
    grid = (Bb, Qp // qblk)
    qspec = pl.BlockSpec((1, qblk, 1), lambda b, i: (b, i, 0))
    cspec = pl.BlockSpec((1, 1, Ncp), lambda b, i: (b, 0, 0))
    ins = [q.reshape(Bb, Qp, 1) for q in qplanes] + \
          [c.reshape(Bb, 1, Ncp) for c in cplanes] + [f_src, x_skip]
    in_specs = [qspec] * 3 + [cspec] * 3 + [
        pl.BlockSpec((1, Ncp, c_src), lambda b, i: (b, 0, 0)),
        pl.BlockSpec((1, qblk, Cs), lambda b, i: (b, i, 0)),
        pl.BlockSpec(Wa.shape, lambda b, i: (0, 0)),
        pl.BlockSpec(Wb.shape, lambda b, i: (0, 0)),
        pl.BlockSpec(b1.shape, lambda b, i: (0,)),
    ]
    args = ins + [Wa, Wb, b1]
    for (W, b) in restps:
        in_specs += [pl.BlockSpec(W.shape, lambda b, i: (0, 0)),
                     pl.BlockSpec(b.shape, lambda b, i: (0,))]
        args += [W, b]
    Cout = restps[-1][0].shape[1]
    return pl.pallas_call(
        functools.partial(_interp_kernel, n_layers=n_layers),
        grid=grid,
        in_specs=in_specs,
        out_specs=pl.BlockSpec((1, qblk, Cout), lambda b, i: (b, i, 0)),
        out_shape=jax.ShapeDtypeStruct((Bb, Qp, Cout), jnp.float32),
    )(*args)


def _head_kernel(f_ref, w0, b0, w1, b1, w2, b2, o_ref):
    h = f_ref[...]
    h = jnp.maximum(h @ w0[...] + b0[...][None, :], 0.0)
    h = jnp.maximum(h @ w1[...] + b1[...][None, :], 0.0)
    o = h @ w2[...] + b2[...][None, :]
    o = o - jax.scipy.special.logsumexp(o, axis=-1, keepdims=True)
    o_ref[...] = o


def _head(f1, ps):
    (w0, b0), (w1, b1), (w2, b2) = ps
    M = f1.shape[0]
    TILE = 2048
    grid = (M // TILE,)
    return pl.pallas_call(
        _head_kernel,
        grid=grid,
        in_specs=[
            pl.BlockSpec((TILE, f1.shape[1]), lambda i: (i, 0)),
            pl.BlockSpec(w0.shape, lambda i: (0, 0)),
            pl.BlockSpec(b0.shape, lambda i: (0,)),
            pl.BlockSpec(w1.shape, lambda i: (0, 0)),
            pl.BlockSpec(b1.shape, lambda i: (0,)),
            pl.BlockSpec(w2.shape, lambda i: (0, 0)),
            pl.BlockSpec(b2.shape, lambda i: (0,)),
        ],
        out_specs=pl.BlockSpec((TILE, NUM_CLASSES), lambda i: (i, 0)),
        out_shape=jax.ShapeDtypeStruct((M, NUM_CLASSES), jnp.float32),
    )(f1, w0, b0, w1, b1, w2, b2)


def _plane_pad(p, npad, fill=PADPOS):
    return jnp.pad(p, ((0, 0), (0, npad - p.shape[1])), constant_values=fill)


def _compact_tmp(keys, tau):
    sel = keys <= tau
    cnt = jnp.sum(sel.astype(jnp.int32), axis=-1)
    _, idx = jax.lax.top_k(-keys, MAX_NEIGH)
    slot = jnp.arange(MAX_NEIGH, dtype=jnp.int32)
    nidx = jnp.where(slot[None, None, :] < cnt[..., None], idx, 0)
    return nidx, cnt


def kernel(x, pos, batch, params):
    del batch
    x0 = x.reshape(B, N, F_IN)
    p0 = pos.reshape(B, N, 3)
    p0x, p0y, p0z = p0[..., 0], p0[..., 1], p0[..., 2]

    _, s1x, s1y, s1z = _fps(p0x, p0y, p0z, N, S1)
    q1x, q1y, q1z = (_plane_pad(s, S1P) for s in (s1x, s1y, s1z))
    keys1, tau1 = _select(q1x, q1y, q1z, p0x, p0y, p0z, RADII[0] ** 2,
                          MAX_NEIGH, qblk=208)
    nidx1, cnt1 = _compact_tmp(keys1, tau1)
    table1 = jnp.concatenate(
        [x0, p0, jnp.zeros((B, N, 10), jnp.float32)], axis=-1).reshape(B * N, 16)
    gid1 = (jnp.arange(B, dtype=jnp.int32)[:, None, None] * N + nidx1).reshape(-1)
    G1 = table1[gid1]
    qpos1 = jnp.stack([q1x, q1y, q1z], axis=-1).reshape(B * S1P, 3)
    x1 = _sa_mlp(G1, qpos1, cnt1.reshape(-1, 1), params['sa1'], MAX_NEIGH,
                 cin_split=3, qblk=104)
    x1 = x1.reshape(B, S1P, 128)

    c1x, c1y, c1z = (_plane_pad(s, NP1) for s in (s1x, s1y, s1z))
    _, s2x, s2y, s2z = _fps(c1x, c1y, c1z, S1, S2)
    q2x, q2y, q2z = (_plane_pad(s, S2P) for s in (s2x, s2y, s2z))
    keys2, tau2 = _select(q2x, q2y, q2z, c1x, c1y, c1z, RADII[1] ** 2,
                          MAX_NEIGH, qblk=256)
    nidx2, cnt2 = _compact_tmp(keys2, tau2)
    x1w = jnp.pad(x1, ((0, 0), (0, NP1 - S1P), (0, 0)))
    table2 = jnp.concatenate(
        [x1w, jnp.stack([c1x, c1y, c1z], axis=-1),
         jnp.zeros((B, NP1, 13), jnp.float32)], axis=-1).reshape(B * NP1, 144)
    gid2 = (jnp.arange(B, dtype=jnp.int32)[:, None, None] * NP1 + nidx2).reshape(-1)
    G2 = table2[gid2]
    qpos2 = jnp.stack([q2x, q2y, q2z], axis=-1).reshape(B * S2P, 3)
    x2 = _sa_mlp(G2, qpos2, cnt2.reshape(-1, 1), params['sa2'], MAX_NEIGH,
                 cin_split=128, qblk=64)
    x2 = x2.reshape(B, S2P, 256)

    pos2 = jnp.stack([q2x, q2y, q2z], axis=-1)
    f3 = _sa3_fp3(x2, pos2, params['sa3'], params['fp3'], S2)

    f2 = _interp_mlp((q1x, q1y, q1z), (q2x, q2y, q2z), f3, x1,
                     params['fp2'], c_src=256, qblk=208)

    f2w = jnp.pad(f2, ((0, 0), (0, NP1 - S1P), (0, 0)))
    f1 = _interp_mlp((p0x, p0y, p0z), (c1x, c1y, c1z), f2w, x0,
                     params['fp1'], c_src=128, qblk=512)

    out = _head(f1.reshape(B * N, 128), params['out'])
    return out.reshape(B * N, NUM_CLASSES)


# speedup vs baseline: 3.7291x; 1.0004x over previous
import functools

import jax
import jax.numpy as jnp
import numpy as np
from jax import lax
from jax.experimental import pallas as pl
from jax.experimental.pallas import tpu as pltpu
from jax.experimental.pallas import tpu_sc as plsc

B = 8
N = 4096
F_IN = 3
NUM_CLASSES = 13
RATIOS = (0.2, 0.25)
RADII = (0.2, 0.4)
MAX_NEIGH = 64


S1, S1P = 819, 832
S2, S2P = 205, 256
NP1 = 896
PADPOS = 1e9


def _tile_store(ref, t, val_col):
    Bb = val_col.shape[0]
    tbase = pl.multiple_of((t // 128) * 128, 128)
    lane = jax.lax.broadcasted_iota(jnp.int32, (Bb, 128), 1)
    sel = lane == (t % 128)
    cur = ref[:, pl.ds(tbase, 128)]
    ref[:, pl.ds(tbase, 128)] = jnp.where(sel, jnp.broadcast_to(val_col, (Bb, 128)), cur)


def _fps_kernel(px_ref, py_ref, pz_ref, d0_ref, idx_ref, sx_ref, sy_ref, sz_ref,
                *, n_samples):
    posx = px_ref[...]
    posy = py_ref[...]
    posz = pz_ref[...]
    np_lanes = posx.shape[1]
    Bb = posx.shape[0]
    lane = jax.lax.broadcasted_iota(jnp.int32, posx.shape, 1)
    idx_ref[...] = jnp.zeros(idx_ref.shape, jnp.int32)
    sx_ref[...] = jnp.zeros(sx_ref.shape, jnp.float32)
    sy_ref[...] = jnp.zeros(sy_ref.shape, jnp.float32)
    sz_ref[...] = jnp.zeros(sz_ref.shape, jnp.float32)

    def step(t, carry):
        last, dists = carry
        onehot = lane == last
        px = jnp.sum(jnp.where(onehot, posx, 0.0), axis=1, keepdims=True)
        py = jnp.sum(jnp.where(onehot, posy, 0.0), axis=1, keepdims=True)
        pz = jnp.sum(jnp.where(onehot, posz, 0.0), axis=1, keepdims=True)
        dx = posx - px
        dy = posy - py
        dz = posz - pz
        d2 = (dx * dx + dy * dy) + dz * dz
        dists = jnp.minimum(dists, d2)
        m = jnp.max(dists, axis=1, keepdims=True)
        cand = jnp.where(dists == m, lane, np_lanes)
        nxt = jnp.min(cand, axis=1, keepdims=True)
        _tile_store(idx_ref, t, nxt)
        _tile_store(sx_ref, t - 1, px)
        _tile_store(sy_ref, t - 1, py)
        _tile_store(sz_ref, t - 1, pz)
        return nxt, dists

    last, _ = jax.lax.fori_loop(
        1, n_samples, step,
        (jnp.zeros((Bb, 1), jnp.int32), d0_ref[...]))
    onehot = lane == last
    _tile_store(sx_ref, n_samples - 1,
                jnp.sum(jnp.where(onehot, posx, 0.0), axis=1, keepdims=True))
    _tile_store(sy_ref, n_samples - 1,
                jnp.sum(jnp.where(onehot, posy, 0.0), axis=1, keepdims=True))
    _tile_store(sz_ref, n_samples - 1,
                jnp.sum(jnp.where(onehot, posz, 0.0), axis=1, keepdims=True))


def _fps(posx, posy, posz, n_real, n_samples):
    Bb, Np = posx.shape
    sp = -n_samples % 128 + n_samples
    lane = jax.lax.broadcasted_iota(jnp.int32, (Bb, Np), 1)
    d0 = jnp.where(lane < n_real, jnp.inf, -jnp.inf).astype(jnp.float32)
    out_shapes = (
        jax.ShapeDtypeStruct((Bb, sp), jnp.int32),
        jax.ShapeDtypeStruct((Bb, sp), jnp.float32),
        jax.ShapeDtypeStruct((Bb, sp), jnp.float32),
        jax.ShapeDtypeStruct((Bb, sp), jnp.float32),
    )
    idx, sx, sy, sz = pl.pallas_call(
        functools.partial(_fps_kernel, n_samples=n_samples),
        out_shape=out_shapes,
    )(posx, posy, posz, d0)
    return (idx[:, :n_samples], sx[:, :n_samples], sy[:, :n_samples],
            sz[:, :n_samples])


_SENT = 0x7F000000


def _select_kernel(qx_ref, qy_ref, qz_ref, px_ref, py_ref, pz_ref,
                   keys_ref, tau_ref, *, r2, k):
    qx = qx_ref[0]
    qy = qy_ref[0]
    qz = qz_ref[0]
    px = px_ref[0]
    py = py_ref[0]
    pz = pz_ref[0]
    dx = qx - px
    dy = qy - py
    dz = qz - pz
    d2 = (dx * dx + dy * dy) + dz * dz
    lane = jax.lax.broadcasted_iota(jnp.int32, d2.shape, 1)
    bits = jax.lax.bitcast_convert_type(d2, jnp.int32)
    keys = jnp.where(d2 <= r2, (bits & (~0xFFF)) | lane, _SENT)
    keys_ref[0] = keys

    q = d2.shape[0]
    lo = jnp.full((q, 1), -1, jnp.int32)
    hi = jnp.full((q, 1), _SENT, jnp.int32)

    def body(_, carry):
        lo, hi = carry
        mid = lo + (hi - lo) // 2
        cnt = jnp.sum((keys <= mid).astype(jnp.int32), axis=1, keepdims=True)
        pred = cnt >= k
        return jnp.where(pred, lo, mid), jnp.where(pred, mid, hi)

    lo, hi = jax.lax.fori_loop(0, 31, body, (lo, hi))
    tau_ref[0] = jnp.minimum(hi, _SENT - 1)


def _select(qx, qy, qz, px, py, pz, r2, k, qblk):
    Bb, Qp = qx.shape
    Np = px.shape[1]
    q3 = qx.reshape(Bb, Qp, 1)
    grid = (Bb, Qp // qblk)
    qspec = pl.BlockSpec((1, qblk, 1), lambda b, i: (b, i, 0))
    pspec = pl.BlockSpec((1, 1, Np), lambda b, i: (b, 0, 0))
    return pl.pallas_call(
        functools.partial(_select_kernel, r2=r2, k=k),
        grid=grid,
        in_specs=[qspec, qspec, qspec, pspec, pspec, pspec],
        out_specs=[
            pl.BlockSpec((1, qblk, Np), lambda b, i: (b, i, 0)),
            pl.BlockSpec((1, qblk, 1), lambda b, i: (b, i, 0)),
        ],
        out_shape=[
            jax.ShapeDtypeStruct((Bb, Qp, Np), jnp.int32),
            jax.ShapeDtypeStruct((Bb, Qp, 1), jnp.int32),
        ],
    )(qx.reshape(Bb, Qp, 1), qy.reshape(Bb, Qp, 1), qz.reshape(Bb, Qp, 1),
      px.reshape(Bb, 1, Np), py.reshape(Bb, 1, Np), pz.reshape(Bb, 1, Np))


def _sa_mlp_kernel(g_ref, qpr_ref, cntr_ref, cnt_ref, w1_ref, w1p_ref, b1_ref,
                   w2_ref, b2_ref, w3_ref, b3_ref, o_ref, *, nneigh):
    g = g_ref[...]
    rows = g.shape[0]
    qb = rows // nneigh
    tq = jnp.dot(qpr_ref[...], w1p_ref[...], preferred_element_type=jnp.float32)
    h = jnp.dot(g, w1_ref[...], preferred_element_type=jnp.float32)
    h = jnp.maximum(h + b1_ref[...][None, :] - tq, 0.0)
    h = jnp.maximum(jnp.dot(h, w2_ref[...], preferred_element_type=jnp.float32)
                    + b2_ref[...][None, :], 0.0)
    h = jnp.dot(h, w3_ref[...], preferred_element_type=jnp.float32) + b3_ref[...][None, :]
    slot = jax.lax.broadcasted_iota(jnp.int32, (rows, 1), 0) % nneigh
    h = jnp.where(slot < cntr_ref[...], h, -jnp.inf)
    m = jnp.max(h.reshape(qb, nneigh, h.shape[-1]), axis=1)
    o_ref[...] = jnp.where(cnt_ref[...] > 0, m, 0.0)


def _sa_mlp(G, qpos, counts, ps, nneigh, cin_split, qblk):
    (W1, b1), (W2, b2), (W3, b3) = ps
    Q, Cpad = G.shape[0] // nneigh, G.shape[1]
    W1x = W1[:cin_split]
    W1p = W1[cin_split:cin_split + 3]
    W1pad = jnp.zeros((Cpad, W1.shape[1]), jnp.float32)
    W1pad = W1pad.at[:cin_split].set(W1x).at[cin_split:cin_split + 3].set(W1p)
    qpos_rep = jnp.broadcast_to(qpos[:, None, :], (Q, nneigh, 3)).reshape(Q * nneigh, 3)
    cnt_rep = jnp.broadcast_to(counts[:, None, :], (Q, nneigh, 1)).reshape(Q * nneigh, 1)
    grid = (Q // qblk,)
    return pl.pallas_call(
        functools.partial(_sa_mlp_kernel, nneigh=nneigh),
        grid=grid,
        in_specs=[
            pl.BlockSpec((qblk * nneigh, Cpad), lambda i: (i, 0)),
            pl.BlockSpec((qblk * nneigh, 3), lambda i: (i, 0)),
            pl.BlockSpec((qblk * nneigh, 1), lambda i: (i, 0)),
            pl.BlockSpec((qblk, 1), lambda i: (i, 0)),
            pl.BlockSpec(W1pad.shape, lambda i: (0, 0)),
            pl.BlockSpec(W1p.shape, lambda i: (0, 0)),
            pl.BlockSpec(b1.shape, lambda i: (0,)),
            pl.BlockSpec(W2.shape, lambda i: (0, 0)),
            pl.BlockSpec(b2.shape, lambda i: (0,)),
            pl.BlockSpec(W3.shape, lambda i: (0, 0)),
            pl.BlockSpec(b3.shape, lambda i: (0,)),
        ],
        out_specs=pl.BlockSpec((qblk, W3.shape[1]), lambda i: (i, 0)),
        out_shape=jax.ShapeDtypeStruct((Q, W3.shape[1]), jnp.float32),
    )(G, qpos_rep, cnt_rep, counts, W1pad, W1p, b1, W2, b2, W3, b3)


def _sa3_fp3_kernel(x2_ref, qp_ref, w1_ref, w1p_ref, b1_ref, w2_ref, b2_ref,
                    w3_ref, b3_ref, f1a_ref, f1b_ref, fb1_ref, f2w_ref,
                    fb2_ref, o_ref, *, n_real):
    x2 = x2_ref[0]
    qp = qp_ref[0]
    h = jnp.dot(x2, w1_ref[...], preferred_element_type=jnp.float32)
    h = h + jnp.dot(qp, w1p_ref[...], preferred_element_type=jnp.float32)
    h = jnp.maximum(h + b1_ref[...][None, :], 0.0)
    h = jnp.maximum(jnp.dot(h, w2_ref[...], preferred_element_type=jnp.float32)
                    + b2_ref[...][None, :], 0.0)
    h = jnp.dot(h, w3_ref[...], preferred_element_type=jnp.float32) + b3_ref[...][None, :]
    row = jax.lax.broadcasted_iota(jnp.int32, h.shape, 0)
    h = jnp.where(row < n_real, h, -jnp.inf)
    xg = jnp.max(h, axis=0, keepdims=True)
    f = jnp.dot(x2, f1b_ref[...], preferred_element_type=jnp.float32)
    f = f + jnp.dot(xg, f1a_ref[...], preferred_element_type=jnp.float32)
    f = jnp.maximum(f + fb1_ref[...][None, :], 0.0)
    f = jnp.dot(f, f2w_ref[...], preferred_element_type=jnp.float32) + fb2_ref[...][None, :]
    row2 = jax.lax.broadcasted_iota(jnp.int32, f.shape, 0)
    o_ref[0] = jnp.where(row2 < n_real, f, 0.0)


def _sa3_fp3(x2, pos2, ps3, psf, n_real):
    (W1, b1), (W2, b2), (W3, b3) = ps3
    (F1, fb1), (F2, fb2) = psf
    Bb, S2p, C = x2.shape
    W1x = W1[:C]
    W1p = W1[C:C + 3]
    F1a = F1[:W3.shape[1]]
    F1b = F1[W3.shape[1]:]
    grid = (Bb,)
    return pl.pallas_call(
        functools.partial(_sa3_fp3_kernel, n_real=n_real),
        grid=grid,
        in_specs=[
            pl.BlockSpec((1, S2p, C), lambda b: (b, 0, 0)),
            pl.BlockSpec((1, S2p, 3), lambda b: (b, 0, 0)),
            pl.BlockSpec(W1x.shape, lambda b: (0, 0)),
            pl.BlockSpec(W1p.shape, lambda b: (0, 0)),
            pl.BlockSpec(b1.shape, lambda b: (0,)),
            pl.BlockSpec(W2.shape, lambda b: (0, 0)),
            pl.BlockSpec(b2.shape, lambda b: (0,)),
            pl.BlockSpec(W3.shape, lambda b: (0, 0)),
            pl.BlockSpec(b3.shape, lambda b: (0,)),
            pl.BlockSpec(F1a.shape, lambda b: (0, 0)),
            pl.BlockSpec(F1b.shape, lambda b: (0, 0)),
            pl.BlockSpec(fb1.shape, lambda b: (0,)),
            pl.BlockSpec(F2.shape, lambda b: (0, 0)),
            pl.BlockSpec(fb2.shape, lambda b: (0,)),
        ],
        out_specs=pl.BlockSpec((1, S2p, F2.shape[1]), lambda b: (b, 0, 0)),
        out_shape=jax.ShapeDtypeStruct((Bb, S2p, F2.shape[1]), jnp.float32),
    )(x2, pos2, W1x, W1p, b1, W2, b2, W3, b3, F1a, F1b, fb1, F2, fb2)


def _interp_kernel(qx_ref, qy_ref, qz_ref, px_ref, py_ref, pz_ref, f_ref,
                   xskip_ref, wa_ref, wb_ref, b1_ref, *rest, n_layers):
    if n_layers == 2:
        w2_ref, b2_ref, o_ref = rest
    else:
        w2_ref, b2_ref, w3_ref, b3_ref, o_ref = rest
    qx = qx_ref[0]
    qy = qy_ref[0]
    qz = qz_ref[0]
    px = px_ref[0]
    py = py_ref[0]
    pz = pz_ref[0]
    dx = qx - px
    dy = qy - py
    dz = qz - pz
    d2 = (dx * dx + dy * dy) + dz * dz
    lane = jax.lax.broadcasted_iota(jnp.int32, d2.shape, 1)
    qb, ncand = d2.shape
    Wint = jnp.zeros(d2.shape, jnp.float32)
    wsum = jnp.zeros((qb, 1), jnp.float32)
    for _ in range(3):
        m = jnp.min(d2, axis=1, keepdims=True)
        cand = jnp.where(d2 == m, lane, ncand)
        idxk = jnp.min(cand, axis=1, keepdims=True)
        w = 1.0 / jnp.maximum(m, 1e-16)
        hit = lane == idxk
        Wint = jnp.where(hit, w, Wint)
        wsum = wsum + w
        d2 = jnp.where(hit, jnp.inf, d2)
    Wint = Wint / wsum
    xi = jnp.dot(Wint, f_ref[0], preferred_element_type=jnp.float32)
    h = jnp.dot(xi, wa_ref[...], preferred_element_type=jnp.float32)
    h = h + jnp.dot(xskip_ref[0], wb_ref[...], preferred_element_type=jnp.float32)
    h = jnp.maximum(h + b1_ref[...][None, :], 0.0)
    h = jnp.dot(h, w2_ref[...], preferred_element_type=jnp.float32) + b2_ref[...][None, :]
    if n_layers == 3:
        h = jnp.maximum(h, 0.0)
        h = jnp.dot(h, w3_ref[...], preferred_element_type=jnp.float32) + b3_ref[...][None, :]
    o_ref[0] = h


def _interp_mlp(qplanes, cplanes, f_src, x_skip, ps, c_src, qblk):
    (W1, b1), *restps = ps
    Bb, Qp = qplanes[0].shape
    Ncp = cplanes[0].shape[1]
    Cs = x_skip.shape[-1]
    Wa = W1[:c_src]
    Wb = W1[c_src:]
    n_layers = 1 + len(restps)
    grid = (Bb, Qp // qblk)
    qspec = pl.BlockSpec((1, qblk, 1), lambda b, i: (b, i, 0))
    cspec = pl.BlockSpec((1, 1, Ncp), lambda b, i: (b, 0, 0))
    ins = [q.reshape(Bb, Qp, 1) for q in qplanes] + \
          [c.reshape(Bb, 1, Ncp) for c in cplanes] + [f_src, x_skip]
    in_specs = [qspec] * 3 + [cspec] * 3 + [
        pl.BlockSpec((1, Ncp, c_src), lambda b, i: (b, 0, 0)),
        pl.BlockSpec((1, qblk, Cs), lambda b, i: (b, i, 0)),
        pl.BlockSpec(Wa.shape, lambda b, i: (0, 0)),
        pl.BlockSpec(Wb.shape, lambda b, i: (0, 0)),
        pl.BlockSpec(b1.shape, lambda b, i: (0,)),
    ]
    args = ins + [Wa, Wb, b1]
    for (W, b) in restps:
        in_specs += [pl.BlockSpec(W.shape, lambda b, i: (0, 0)),
                     pl.BlockSpec(b.shape, lambda b, i: (0,))]
        args += [W, b]
    Cout = restps[-1][0].shape[1]
    return pl.pallas_call(
        functools.partial(_interp_kernel, n_layers=n_layers),
        grid=grid,
        in_specs=in_specs,
        out_specs=pl.BlockSpec((1, qblk, Cout), lambda b, i: (b, i, 0)),
        out_shape=jax.ShapeDtypeStruct((Bb, Qp, Cout), jnp.float32),
    )(*args)


def _head_kernel(f_ref, w0, b0, w1, b1, w2, b2, o_ref):
    h = f_ref[...]
    h = jnp.maximum(h @ w0[...] + b0[...][None, :], 0.0)
    h = jnp.maximum(h @ w1[...] + b1[...][None, :], 0.0)
    o = h @ w2[...] + b2[...][None, :]
    o = o - jax.scipy.special.logsumexp(o, axis=-1, keepdims=True)
    o_ref[...] = o


def _head(f1, ps):
    (w0, b0), (w1, b1), (w2, b2) = ps
    M = f1.shape[0]
    TILE = 2048
    grid = (M // TILE,)
    return pl.pallas_call(
        _head_kernel,
        grid=grid,
        in_specs=[
            pl.BlockSpec((TILE, f1.shape[1]), lambda i: (i, 0)),
            pl.BlockSpec(w0.shape, lambda i: (0, 0)),
            pl.BlockSpec(b0.shape, lambda i: (0,)),
            pl.BlockSpec(w1.shape, lambda i: (0, 0)),
            pl.BlockSpec(b1.shape, lambda i: (0,)),
            pl.BlockSpec(w2.shape, lambda i: (0, 0)),
            pl.BlockSpec(b2.shape, lambda i: (0,)),
        ],
        out_specs=pl.BlockSpec((TILE, NUM_CLASSES), lambda i: (i, 0)),
        out_shape=jax.ShapeDtypeStruct((M, NUM_CLASSES), jnp.float32),
    )(f1, w0, b0, w1, b1, w2, b2)


_SC_TILES = 32


def _sc_compact_gather(keys, tau, table, rows_per_b, cand_pb):
    R, Np = keys.shape
    T, C = table.shape
    NR = R // _SC_TILES
    n_chunks = Np // 16
    tau16 = jnp.broadcast_to(tau[:, None], (R, 16))
    mesh = plsc.VectorSubcoreMesh(core_axis_name="c", subcore_axis_name="s")

    @functools.partial(
        pl.kernel, mesh=mesh,
        out_type=[
            jax.ShapeDtypeStruct((R * MAX_NEIGH, C), jnp.float32),
            jax.ShapeDtypeStruct((R * 16,), jnp.int32),
        ],
        scratch_types=[
            pltpu.VMEM((16,), jnp.int32),
            pltpu.VMEM((Np,), jnp.int32),
            pltpu.VMEM((MAX_NEIGH,), jnp.int32),
            pltpu.VMEM((MAX_NEIGH, C), jnp.float32),
            pltpu.VMEM((NR * 16,), jnp.int32),
            pltpu.SemaphoreType.DMA,
        ],
    )
    def k(keys_hbm, tau_hbm, table_hbm, g_out, cnt_out,
          tau_v, kbuf, idxbuf, gbuf, cnt_v, sem):
        wid = lax.axis_index("s") * 2 + lax.axis_index("c")
        base = wid * NR
        iota = lax.iota(jnp.int32, 16)
        for c4 in range(MAX_NEIGH // 16):
            idxbuf[pl.ds(c4 * 16, 16)] = jnp.zeros((16,), jnp.int32)

        def row_body(rloc, _):
            r = base + rloc
            gbase = (r // rows_per_b) * cand_pb
            pltpu.sync_copy(keys_hbm.at[r], kbuf)
            pltpu.sync_copy(tau_hbm.at[r], tau_v)
            t16 = tau_v[...]

            def chunk(c, off):
                kk = kbuf[pl.ds(c * 16, 16)]
                m = kk <= t16
                mi = m.astype(jnp.int32)
                cum = plsc.cumsum(mi)
                pos = cum + (off - 1)
                vals = iota + (gbase + c * 16)
                plsc.store_scatter(idxbuf, [pos], vals, mask=m)
                return off + jnp.sum(mi)

            off = lax.fori_loop(0, n_chunks, chunk, jnp.int32(0))
            cnt_v[pl.ds(rloc * 16, 16)] = jnp.zeros((16,), jnp.int32) + off
            pltpu.async_copy(table_hbm.at[idxbuf], gbuf, sem).wait()
            pltpu.sync_copy(gbuf, g_out.at[pl.ds(r * MAX_NEIGH, MAX_NEIGH)])
            return 0

        lax.fori_loop(0, NR, row_body, 0)
        pltpu.sync_copy(cnt_v, cnt_out.at[pl.ds(base * 16, NR * 16)])

    G, cnt16 = k(keys, tau16, table)
    return G, cnt16.reshape(R, 16)[:, 0]


def _tie(dep, *xs):
    out = lax.optimization_barrier((dep, *xs))
    return out[1:] if len(xs) > 1 else out[1]


def _plane_pad(p, npad, fill=PADPOS):
    return jnp.pad(p, ((0, 0), (0, npad - p.shape[1])), constant_values=fill)


def _compact_tmp(keys, tau):
    sel = keys <= tau
    cnt = jnp.sum(sel.astype(jnp.int32), axis=-1)
    _, idx = jax.lax.top_k(-keys, MAX_NEIGH)
    slot = jnp.arange(MAX_NEIGH, dtype=jnp.int32)
    nidx = jnp.where(slot[None, None, :] < cnt[..., None], idx, 0)
    return nidx, cnt


def kernel(x, pos, batch, params):
    del batch
    x0 = x.reshape(B, N, F_IN)
    p0 = pos.reshape(B, N, 3)
    p0x, p0y, p0z = p0[..., 0], p0[..., 1], p0[..., 2]

    _, s1x, s1y, s1z = _fps(p0x, p0y, p0z, N, S1)
    q1x, q1y, q1z = (_plane_pad(s, S1P) for s in (s1x, s1y, s1z))
    keys1, tau1 = _select(q1x, q1y, q1z, p0x, p0y, p0z, RADII[0] ** 2,
                          MAX_NEIGH, qblk=208)
    table1 = jnp.concatenate(
        [x0, p0, jnp.zeros((B, N, 10), jnp.float32)], axis=-1).reshape(B * N, 16)
    nidx1, cnt1 = _compact_tmp(keys1, tau1)
    gid1 = (jnp.arange(B, dtype=jnp.int32)[:, None, None] * N + nidx1).reshape(-1)
    G1 = table1[gid1]
    qpos1 = jnp.stack([q1x, q1y, q1z], axis=-1).reshape(B * S1P, 3)
    x1 = _sa_mlp(G1, qpos1, cnt1.reshape(-1, 1), params['sa1'], MAX_NEIGH,
                 cin_split=3, qblk=104)
    x1 = x1.reshape(B, S1P, 128)

    c1x, c1y, c1z = (_plane_pad(s, NP1) for s in (s1x, s1y, s1z))
    c1x, c1y, c1z = _tie(cnt1, c1x, c1y, c1z)
    _, s2x, s2y, s2z = _fps(c1x, c1y, c1z, S1, S2)
    q2x, q2y, q2z = (_plane_pad(s, S2P) for s in (s2x, s2y, s2z))
    keys2, tau2 = _select(q2x, q2y, q2z, c1x, c1y, c1z, RADII[1] ** 2,
                          MAX_NEIGH, qblk=256)
    x1w = jnp.pad(x1, ((0, 0), (0, NP1 - S1P), (0, 0)))
    table2 = jnp.concatenate(
        [x1w, jnp.stack([c1x, c1y, c1z], axis=-1),
         jnp.zeros((B, NP1, 13), jnp.float32)], axis=-1).reshape(B * NP1, 144)
    nidx2, cnt2 = _compact_tmp(keys2, tau2)
    gid2 = (jnp.arange(B, dtype=jnp.int32)[:, None, None] * NP1 + nidx2).reshape(-1)
    G2 = table2[gid2]
    qpos2 = jnp.stack([q2x, q2y, q2z], axis=-1).reshape(B * S2P, 3)
    x2 = _sa_mlp(G2, qpos2, cnt2.reshape(-1, 1), params['sa2'], MAX_NEIGH,
                 cin_split=128, qblk=64)
    x2 = x2.reshape(B, S2P, 256)

    x2 = _tie(cnt2, x2)
    pos2 = jnp.stack([q2x, q2y, q2z], axis=-1)
    f3 = _sa3_fp3(x2, pos2, params['sa3'], params['fp3'], S2)

    f2 = _interp_mlp((q1x, q1y, q1z), (q2x, q2y, q2z), f3, x1,
                     params['fp2'], c_src=256, qblk=208)

    f2w = jnp.pad(f2, ((0, 0), (0, NP1 - S1P), (0, 0)))
    f1 = _interp_mlp((p0x, p0y, p0z), (c1x, c1y, c1z), f2w, x0,
                     params['fp1'], c_src=128, qblk=512)

    out = _head(f1.reshape(B * N, 128), params['out'])
    return out.reshape(B * N, NUM_CLASSES)


# speedup vs baseline: 4.2902x; 1.1505x over previous
import functools

import jax
import jax.numpy as jnp
import numpy as np
from jax import lax
from jax.experimental import pallas as pl
from jax.experimental.pallas import tpu as pltpu
from jax.experimental.pallas import tpu_sc as plsc

B = 8
N = 4096
F_IN = 3
NUM_CLASSES = 13
RATIOS = (0.2, 0.25)
RADII = (0.2, 0.4)
MAX_NEIGH = 64


S1, S1P = 819, 832
S2, S2P = 205, 256
NP1 = 896
PADPOS = 1e9


def _tile_store(ref, t, val_col):
    Bb = val_col.shape[0]
    tbase = pl.multiple_of((t // 128) * 128, 128)
    lane = jax.lax.broadcasted_iota(jnp.int32, (Bb, 128), 1)
    sel = lane == (t % 128)
    cur = ref[:, pl.ds(tbase, 128)]
    ref[:, pl.ds(tbase, 128)] = jnp.where(sel, jnp.broadcast_to(val_col, (Bb, 128)), cur)


def _fps_kernel(px_ref, py_ref, pz_ref, d0_ref, idx_ref, sx_ref, sy_ref, sz_ref,
                *, n_samples):
    posx = px_ref[...]
    posy = py_ref[...]
    posz = pz_ref[...]
    np_lanes = posx.shape[1]
    Bb = posx.shape[0]
    lane = jax.lax.broadcasted_iota(jnp.int32, posx.shape, 1)
    idx_ref[...] = jnp.zeros(idx_ref.shape, jnp.int32)
    sx_ref[...] = jnp.zeros(sx_ref.shape, jnp.float32)
    sy_ref[...] = jnp.zeros(sy_ref.shape, jnp.float32)
    sz_ref[...] = jnp.zeros(sz_ref.shape, jnp.float32)

    def step(t, carry):
        last, dists = carry
        onehot = lane == last
        px = jnp.sum(jnp.where(onehot, posx, 0.0), axis=1, keepdims=True)
        py = jnp.sum(jnp.where(onehot, posy, 0.0), axis=1, keepdims=True)
        pz = jnp.sum(jnp.where(onehot, posz, 0.0), axis=1, keepdims=True)
        dx = posx - px
        dy = posy - py
        dz = posz - pz
        d2 = (dx * dx + dy * dy) + dz * dz
        dists = jnp.minimum(dists, d2)
        m = jnp.max(dists, axis=1, keepdims=True)
        cand = jnp.where(dists == m, lane, np_lanes)
        nxt = jnp.min(cand, axis=1, keepdims=True)
        _tile_store(idx_ref, t, nxt)
        _tile_store(sx_ref, t - 1, px)
        _tile_store(sy_ref, t - 1, py)
        _tile_store(sz_ref, t - 1, pz)
        return nxt, dists

    last, _ = jax.lax.fori_loop(
        1, n_samples, step,
        (jnp.zeros((Bb, 1), jnp.int32), d0_ref[...]))
    onehot = lane == last
    _tile_store(sx_ref, n_samples - 1,
                jnp.sum(jnp.where(onehot, posx, 0.0), axis=1, keepdims=True))
    _tile_store(sy_ref, n_samples - 1,
                jnp.sum(jnp.where(onehot, posy, 0.0), axis=1, keepdims=True))
    _tile_store(sz_ref, n_samples - 1,
                jnp.sum(jnp.where(onehot, posz, 0.0), axis=1, keepdims=True))


def _fps(posx, posy, posz, n_real, n_samples):
    Bb, Np = posx.shape
    sp = -n_samples % 128 + n_samples
    lane = jax.lax.broadcasted_iota(jnp.int32, (Bb, Np), 1)
    d0 = jnp.where(lane < n_real, jnp.inf, -jnp.inf).astype(jnp.float32)
    out_shapes = (
        jax.ShapeDtypeStruct((Bb, sp), jnp.int32),
        jax.ShapeDtypeStruct((Bb, sp), jnp.float32),
        jax.ShapeDtypeStruct((Bb, sp), jnp.float32),
        jax.ShapeDtypeStruct((Bb, sp), jnp.float32),
    )
    idx, sx, sy, sz = pl.pallas_call(
        functools.partial(_fps_kernel, n_samples=n_samples),
        out_shape=out_shapes,
    )(posx, posy, posz, d0)
    return (idx[:, :n_samples], sx[:, :n_samples], sy[:, :n_samples],
            sz[:, :n_samples])


_SENT = 0x7F000000


def _select_kernel(qx_ref, qy_ref, qz_ref, px_ref, py_ref, pz_ref,
                   keys_ref, tau_ref, *, r2, k):
    qx = qx_ref[0]
    qy = qy_ref[0]
    qz = qz_ref[0]
    px = px_ref[0]
    py = py_ref[0]
    pz = pz_ref[0]
    dx = qx - px
    dy = qy - py
    dz = qz - pz
    d2 = (dx * dx + dy * dy) + dz * dz
    lane = jax.lax.broadcasted_iota(jnp.int32, d2.shape, 1)
    bits = jax.lax.bitcast_convert_type(d2, jnp.int32)
    keys = jnp.where(d2 <= r2, (bits & (~0xFFF)) | lane, _SENT)
    keys_ref[0] = keys

    q = d2.shape[0]
    lo = jnp.full((q, 1), -1, jnp.int32)
    hi = jnp.full((q, 1), _SENT, jnp.int32)

    def body(_, carry):
        lo, hi = carry
        mid = lo + (hi - lo) // 2
        cnt = jnp.sum((keys <= mid).astype(jnp.int32), axis=1, keepdims=True)
        pred = cnt >= k
        return jnp.where(pred, lo, mid), jnp.where(pred, mid, hi)

    lo, hi = jax.lax.fori_loop(0, 31, body, (lo, hi))
    tau_ref[0] = jnp.minimum(hi, _SENT - 1)


def _select(qx, qy, qz, px, py, pz, r2, k, qblk):
    Bb, Qp = qx.shape
    Np = px.shape[1]
    q3 = qx.reshape(Bb, Qp, 1)
    grid = (Bb, Qp // qblk)
    qspec = pl.BlockSpec((1, qblk, 1), lambda b, i: (b, i, 0))
    pspec = pl.BlockSpec((1, 1, Np), lambda b, i: (b, 0, 0))
    return pl.pallas_call(
        functools.partial(_select_kernel, r2=r2, k=k),
        grid=grid,
        in_specs=[qspec, qspec, qspec, pspec, pspec, pspec],
        out_specs=[
            pl.BlockSpec((1, qblk, Np), lambda b, i: (b, i, 0)),
            pl.BlockSpec((1, qblk, 1), lambda b, i: (b, i, 0)),
        ],
        out_shape=[
            jax.ShapeDtypeStruct((Bb, Qp, Np), jnp.int32),
            jax.ShapeDtypeStruct((Bb, Qp, 1), jnp.int32),
        ],
    )(qx.reshape(Bb, Qp, 1), qy.reshape(Bb, Qp, 1), qz.reshape(Bb, Qp, 1),
      px.reshape(Bb, 1, Np), py.reshape(Bb, 1, Np), pz.reshape(Bb, 1, Np))


def _sa_mlp_kernel(g_ref, qpr_ref, cntr_ref, cnt_ref, w1_ref, w1p_ref, b1_ref,
                   w2_ref, b2_ref, w3_ref, b3_ref, o_ref, *, nneigh):
    g = g_ref[...]
    rows = g.shape[0]
    qb = rows // nneigh
    tq = jnp.dot(qpr_ref[...], w1p_ref[...], preferred_element_type=jnp.float32)
    h = jnp.dot(g, w1_ref[...], preferred_element_type=jnp.float32)
    h = jnp.maximum(h + b1_ref[...][None, :] - tq, 0.0)
    h = jnp.maximum(jnp.dot(h, w2_ref[...], preferred_element_type=jnp.float32)
                    + b2_ref[...][None, :], 0.0)
    h = jnp.dot(h, w3_ref[...], preferred_element_type=jnp.float32) + b3_ref[...][None, :]
    h = jnp.where(cntr_ref[...] > 0, h, -jnp.inf)
    m = jnp.max(h.reshape(qb, nneigh, h.shape[-1]), axis=1)
    o_ref[...] = jnp.where(cnt_ref[...] > 0, m, 0.0)


def _sa_mlp(G, qpos, counts, ps, nneigh, cin_split, qblk, vflags):
    (W1, b1), (W2, b2), (W3, b3) = ps
    Q, Cpad = G.shape[0] // nneigh, G.shape[1]
    W1x = W1[:cin_split]
    W1p = W1[cin_split:cin_split + 3]
    W1pad = jnp.zeros((Cpad, W1.shape[1]), jnp.float32)
    W1pad = W1pad.at[:cin_split].set(W1x).at[cin_split:cin_split + 3].set(W1p)
    qpos_rep = jnp.broadcast_to(qpos[:, None, :], (Q, nneigh, 3)).reshape(Q * nneigh, 3)
    cnt_rep = vflags
    grid = (Q // qblk,)
    return pl.pallas_call(
        functools.partial(_sa_mlp_kernel, nneigh=nneigh),
        grid=grid,
        in_specs=[
            pl.BlockSpec((qblk * nneigh, Cpad), lambda i: (i, 0)),
            pl.BlockSpec((qblk * nneigh, 3), lambda i: (i, 0)),
            pl.BlockSpec((qblk * nneigh, 1), lambda i: (i, 0)),
            pl.BlockSpec((qblk, 1), lambda i: (i, 0)),
            pl.BlockSpec(W1pad.shape, lambda i: (0, 0)),
            pl.BlockSpec(W1p.shape, lambda i: (0, 0)),
            pl.BlockSpec(b1.shape, lambda i: (0,)),
            pl.BlockSpec(W2.shape, lambda i: (0, 0)),
            pl.BlockSpec(b2.shape, lambda i: (0,)),
            pl.BlockSpec(W3.shape, lambda i: (0, 0)),
            pl.BlockSpec(b3.shape, lambda i: (0,)),
        ],
        out_specs=pl.BlockSpec((qblk, W3.shape[1]), lambda i: (i, 0)),
        out_shape=jax.ShapeDtypeStruct((Q, W3.shape[1]), jnp.float32),
    )(G, qpos_rep, cnt_rep, counts, W1pad, W1p, b1, W2, b2, W3, b3)


def _sa3_fp3_kernel(x2_ref, qp_ref, w1_ref, w1p_ref, b1_ref, w2_ref, b2_ref,
                    w3_ref, b3_ref, f1a_ref, f1b_ref, fb1_ref, f2w_ref,
                    fb2_ref, o_ref, *, n_real):
    x2 = x2_ref[0]
    qp = qp_ref[0]
    h = jnp.dot(x2, w1_ref[...], preferred_element_type=jnp.float32)
    h = h + jnp.dot(qp, w1p_ref[...], preferred_element_type=jnp.float32)
    h = jnp.maximum(h + b1_ref[...][None, :], 0.0)
    h = jnp.maximum(jnp.dot(h, w2_ref[...], preferred_element_type=jnp.float32)
                    + b2_ref[...][None, :], 0.0)
    h = jnp.dot(h, w3_ref[...], preferred_element_type=jnp.float32) + b3_ref[...][None, :]
    row = jax.lax.broadcasted_iota(jnp.int32, h.shape, 0)
    h = jnp.where(row < n_real, h, -jnp.inf)
    xg = jnp.max(h, axis=0, keepdims=True)
    f = jnp.dot(x2, f1b_ref[...], preferred_element_type=jnp.float32)
    f = f + jnp.dot(xg, f1a_ref[...], preferred_element_type=jnp.float32)
    f = jnp.maximum(f + fb1_ref[...][None, :], 0.0)
    f = jnp.dot(f, f2w_ref[...], preferred_element_type=jnp.float32) + fb2_ref[...][None, :]
    row2 = jax.lax.broadcasted_iota(jnp.int32, f.shape, 0)
    o_ref[0] = jnp.where(row2 < n_real, f, 0.0)


def _sa3_fp3(x2, pos2, ps3, psf, n_real):
    (W1, b1), (W2, b2), (W3, b3) = ps3
    (F1, fb1), (F2, fb2) = psf
    Bb, S2p, C = x2.shape
    W1x = W1[:C]
    W1p = W1[C:C + 3]
    F1a = F1[:W3.shape[1]]
    F1b = F1[W3.shape[1]:]
    grid = (Bb,)
    return pl.pallas_call(
        functools.partial(_sa3_fp3_kernel, n_real=n_real),
        grid=grid,
        in_specs=[
            pl.BlockSpec((1, S2p, C), lambda b: (b, 0, 0)),
            pl.BlockSpec((1, S2p, 3), lambda b: (b, 0, 0)),
            pl.BlockSpec(W1x.shape, lambda b: (0, 0)),
            pl.BlockSpec(W1p.shape, lambda b: (0, 0)),
            pl.BlockSpec(b1.shape, lambda b: (0,)),
            pl.BlockSpec(W2.shape, lambda b: (0, 0)),
            pl.BlockSpec(b2.shape, lambda b: (0,)),
            pl.BlockSpec(W3.shape, lambda b: (0, 0)),
            pl.BlockSpec(b3.shape, lambda b: (0,)),
            pl.BlockSpec(F1a.shape, lambda b: (0, 0)),
            pl.BlockSpec(F1b.shape, lambda b: (0, 0)),
            pl.BlockSpec(fb1.shape, lambda b: (0,)),
            pl.BlockSpec(F2.shape, lambda b: (0, 0)),
            pl.BlockSpec(fb2.shape, lambda b: (0,)),
        ],
        out_specs=pl.BlockSpec((1, S2p, F2.shape[1]), lambda b: (b, 0, 0)),
        out_shape=jax.ShapeDtypeStruct((Bb, S2p, F2.shape[1]), jnp.float32),
    )(x2, pos2, W1x, W1p, b1, W2, b2, W3, b3, F1a, F1b, fb1, F2, fb2)


def _interp_kernel(qx_ref, qy_ref, qz_ref, px_ref, py_ref, pz_ref, f_ref,
                   xskip_ref, wa_ref, wb_ref, b1_ref, *rest, n_layers):
    if n_layers == 2:
        w2_ref, b2_ref, o_ref = rest
    else:
        w2_ref, b2_ref, w3_ref, b3_ref, o_ref = rest
    qx = qx_ref[0]
    qy = qy_ref[0]
    qz = qz_ref[0]
    px = px_ref[0]
    py = py_ref[0]
    pz = pz_ref[0]
    dx = qx - px
    dy = qy - py
    dz = qz - pz
    d2 = (dx * dx + dy * dy) + dz * dz
    lane = jax.lax.broadcasted_iota(jnp.int32, d2.shape, 1)
    qb, ncand = d2.shape
    Wint = jnp.zeros(d2.shape, jnp.float32)
    wsum = jnp.zeros((qb, 1), jnp.float32)
    for _ in range(3):
        m = jnp.min(d2, axis=1, keepdims=True)
        cand = jnp.where(d2 == m, lane, ncand)
        idxk = jnp.min(cand, axis=1, keepdims=True)
        w = 1.0 / jnp.maximum(m, 1e-16)
        hit = lane == idxk
        Wint = jnp.where(hit, w, Wint)
        wsum = wsum + w
        d2 = jnp.where(hit, jnp.inf, d2)
    Wint = Wint / wsum
    xi = jnp.dot(Wint, f_ref[0], preferred_element_type=jnp.float32)
    h = jnp.dot(xi, wa_ref[...], preferred_element_type=jnp.float32)
    h = h + jnp.dot(xskip_ref[0], wb_ref[...], preferred_element_type=jnp.float32)
    h = jnp.maximum(h + b1_ref[...][None, :], 0.0)
    h = jnp.dot(h, w2_ref[...], preferred_element_type=jnp.float32) + b2_ref[...][None, :]
    if n_layers == 3:
        h = jnp.maximum(h, 0.0)
        h = jnp.dot(h, w3_ref[...], preferred_element_type=jnp.float32) + b3_ref[...][None, :]
    o_ref[0] = h


def _interp_mlp(qplanes, cplanes, f_src, x_skip, ps, c_src, qblk):
    (W1, b1), *restps = ps
    Bb, Qp = qplanes[0].shape
    Ncp = cplanes[0].shape[1]
    Cs = x_skip.shape[-1]
    Wa = W1[:c_src]
    Wb = W1[c_src:]
    n_layers = 1 + len(restps)
    grid = (Bb, Qp // qblk)
    qspec = pl.BlockSpec((1, qblk, 1), lambda b, i: (b, i, 0))
    cspec = pl.BlockSpec((1, 1, Ncp), lambda b, i: (b, 0, 0))
    ins = [q.reshape(Bb, Qp, 1) for q in qplanes] + \
          [c.reshape(Bb, 1, Ncp) for c in cplanes] + [f_src, x_skip]
    in_specs = [qspec] * 3 + [cspec] * 3 + [
        pl.BlockSpec((1, Ncp, c_src), lambda b, i: (b, 0, 0)),
        pl.BlockSpec((1, qblk, Cs), lambda b, i: (b, i, 0)),
        pl.BlockSpec(Wa.shape, lambda b, i: (0, 0)),
        pl.BlockSpec(Wb.shape, lambda b, i: (0, 0)),
        pl.BlockSpec(b1.shape, lambda b, i: (0,)),
    ]
    args = ins + [Wa, Wb, b1]
    for (W, b) in restps:
        in_specs += [pl.BlockSpec(W.shape, lambda b, i: (0, 0)),
                     pl.BlockSpec(b.shape, lambda b, i: (0,))]
        args += [W, b]
    Cout = restps[-1][0].shape[1]
    return pl.pallas_call(
        functools.partial(_interp_kernel, n_layers=n_layers),
        grid=grid,
        in_specs=in_specs,
        out_specs=pl.BlockSpec((1, qblk, Cout), lambda b, i: (b, i, 0)),
        out_shape=jax.ShapeDtypeStruct((Bb, Qp, Cout), jnp.float32),
    )(*args)


def _head_kernel(f_ref, w0, b0, w1, b1, w2, b2, o_ref):
    h = f_ref[...]
    h = jnp.maximum(h @ w0[...] + b0[...][None, :], 0.0)
    h = jnp.maximum(h @ w1[...] + b1[...][None, :], 0.0)
    o = h @ w2[...] + b2[...][None, :]
    o = o - jax.scipy.special.logsumexp(o, axis=-1, keepdims=True)
    o_ref[...] = o


def _head(f1, ps):
    (w0, b0), (w1, b1), (w2, b2) = ps
    M = f1.shape[0]
    TILE = 2048
    grid = (M // TILE,)
    return pl.pallas_call(
        _head_kernel,
        grid=grid,
        in_specs=[
            pl.BlockSpec((TILE, f1.shape[1]), lambda i: (i, 0)),
            pl.BlockSpec(w0.shape, lambda i: (0, 0)),
            pl.BlockSpec(b0.shape, lambda i: (0,)),
            pl.BlockSpec(w1.shape, lambda i: (0, 0)),
            pl.BlockSpec(b1.shape, lambda i: (0,)),
            pl.BlockSpec(w2.shape, lambda i: (0, 0)),
            pl.BlockSpec(b2.shape, lambda i: (0,)),
        ],
        out_specs=pl.BlockSpec((TILE, NUM_CLASSES), lambda i: (i, 0)),
        out_shape=jax.ShapeDtypeStruct((M, NUM_CLASSES), jnp.float32),
    )(f1, w0, b0, w1, b1, w2, b2)


_SC_TILES = 32


def _sc_compact_gather(keys, tau, table, rows_per_b, cand_pb):
    R, Np = keys.shape
    T, C = table.shape
    NR = R // _SC_TILES
    n_chunks = Np // 16
    tau16 = jnp.broadcast_to(tau[:, None], (R, 16))
    mesh = plsc.VectorSubcoreMesh(core_axis_name="c", subcore_axis_name="s")

    @functools.partial(
        pl.kernel, mesh=mesh,
        out_type=[
            jax.ShapeDtypeStruct((R * MAX_NEIGH, C), jnp.float32),
            jax.ShapeDtypeStruct((R * 16,), jnp.int32),
        ],
        scratch_types=[
            pltpu.VMEM((16,), jnp.int32),
            pltpu.VMEM((Np,), jnp.int32),
            pltpu.VMEM((MAX_NEIGH,), jnp.int32),
            pltpu.VMEM((MAX_NEIGH, C), jnp.float32),
            pltpu.VMEM((NR * 16,), jnp.int32),
            pltpu.SemaphoreType.DMA,
        ],
    )
    def k(keys_hbm, tau_hbm, table_hbm, g_out, cnt_out,
          tau_v, kbuf, idxbuf, gbuf, cnt_v, sem):
        wid = lax.axis_index("s") * 2 + lax.axis_index("c")
        base = wid * NR
        iota = lax.iota(jnp.int32, 16)
        for c4 in range(MAX_NEIGH // 16):
            idxbuf[pl.ds(c4 * 16, 16)] = jnp.zeros((16,), jnp.int32)

        def row_body(rloc, _):
            r = base + rloc
            gbase = (r // rows_per_b) * cand_pb
            pltpu.sync_copy(keys_hbm.at[r], kbuf)
            pltpu.sync_copy(tau_hbm.at[r], tau_v)
            t16 = tau_v[...]

            def chunk(c, off):
                kk = kbuf[pl.ds(c * 16, 16)]
                m = kk <= t16
                mi = m.astype(jnp.int32)
                cum = plsc.cumsum(mi)
                pos = cum + (off - 1)
                vals = iota + (gbase + c * 16)
                plsc.store_scatter(idxbuf, [pos], vals, mask=m)
                return off + jnp.sum(mi)

            off = lax.fori_loop(0, n_chunks, chunk, jnp.int32(0))
            cnt_v[pl.ds(rloc * 16, 16)] = jnp.zeros((16,), jnp.int32) + off
            pltpu.async_copy(table_hbm.at[idxbuf], gbuf, sem).wait()
            pltpu.sync_copy(gbuf, g_out.at[pl.ds(r * MAX_NEIGH, MAX_NEIGH)])
            return 0

        lax.fori_loop(0, NR, row_body, 0)
        pltpu.sync_copy(cnt_v, cnt_out.at[pl.ds(base * 16, NR * 16)])

    G, cnt16 = k(keys, tau16, table)
    return G, cnt16.reshape(R, 16)[:, 0]


def _tie(dep, *xs):
    out = lax.optimization_barrier((dep, *xs))
    return out[1:] if len(xs) > 1 else out[1]


def _plane_pad(p, npad, fill=PADPOS):
    return jnp.pad(p, ((0, 0), (0, npad - p.shape[1])), constant_values=fill)


def _compact_tmp(keys, tau):
    _, nidx = jax.lax.approx_min_k(keys.astype(jnp.float32), MAX_NEIGH,
                                   recall_target=0.99)
    nidx = nidx.astype(jnp.int32)
    kg = jnp.take_along_axis(keys, nidx, axis=-1)
    valid = (kg <= tau).astype(jnp.int32)
    cnt = jnp.sum(valid, axis=-1)
    nidx = jnp.where(valid > 0, nidx, 0)
    return nidx, valid, cnt


def kernel(x, pos, batch, params):
    del batch
    x0 = x.reshape(B, N, F_IN)
    p0 = pos.reshape(B, N, 3)
    p0x, p0y, p0z = p0[..., 0], p0[..., 1], p0[..., 2]

    _, s1x, s1y, s1z = _fps(p0x, p0y, p0z, N, S1)
    q1x, q1y, q1z = (_plane_pad(s, S1P) for s in (s1x, s1y, s1z))
    keys1, tau1 = _select(q1x, q1y, q1z, p0x, p0y, p0z, RADII[0] ** 2,
                          MAX_NEIGH, qblk=208)
    table1 = jnp.concatenate(
        [x0, p0, jnp.zeros((B, N, 10), jnp.float32)], axis=-1).reshape(B * N, 16)
    nidx1, val1, cnt1 = _compact_tmp(keys1, tau1)
    gid1 = (jnp.arange(B, dtype=jnp.int32)[:, None, None] * N + nidx1).reshape(-1)
    G1 = table1[gid1]
    qpos1 = jnp.stack([q1x, q1y, q1z], axis=-1).reshape(B * S1P, 3)
    x1 = _sa_mlp(G1, qpos1, cnt1.reshape(-1, 1), params['sa1'], MAX_NEIGH,
                 cin_split=3, qblk=104, vflags=val1.reshape(-1, 1))
    x1 = x1.reshape(B, S1P, 128)

    c1x, c1y, c1z = (_plane_pad(s, NP1) for s in (s1x, s1y, s1z))
    c1x, c1y, c1z = _tie(cnt1, c1x, c1y, c1z)
    _, s2x, s2y, s2z = _fps(c1x, c1y, c1z, S1, S2)
    q2x, q2y, q2z = (_plane_pad(s, S2P) for s in (s2x, s2y, s2z))
    keys2, tau2 = _select(q2x, q2y, q2z, c1x, c1y, c1z, RADII[1] ** 2,
                          MAX_NEIGH, qblk=256)
    x1w = jnp.pad(x1, ((0, 0), (0, NP1 - S1P), (0, 0)))
    table2 = jnp.concatenate(
        [x1w, jnp.stack([c1x, c1y, c1z], axis=-1),
         jnp.zeros((B, NP1, 13), jnp.float32)], axis=-1).reshape(B * NP1, 144)
    nidx2, val2, cnt2 = _compact_tmp(keys2, tau2)
    gid2 = (jnp.arange(B, dtype=jnp.int32)[:, None, None] * NP1 + nidx2).reshape(-1)
    G2 = table2[gid2]
    qpos2 = jnp.stack([q2x, q2y, q2z], axis=-1).reshape(B * S2P, 3)
    x2 = _sa_mlp(G2, qpos2, cnt2.reshape(-1, 1), params['sa2'], MAX_NEIGH,
                 cin_split=128, qblk=64, vflags=val2.reshape(-1, 1))
    x2 = x2.reshape(B, S2P, 256)

    x2 = _tie(cnt2, x2)
    pos2 = jnp.stack([q2x, q2y, q2z], axis=-1)
    f3 = _sa3_fp3(x2, pos2, params['sa3'], params['fp3'], S2)

    f2 = _interp_mlp((q1x, q1y, q1z), (q2x, q2y, q2z), f3, x1,
                     params['fp2'], c_src=256, qblk=208)

    f2w = jnp.pad(f2, ((0, 0), (0, NP1 - S1P), (0, 0)))
    f1 = _interp_mlp((p0x, p0y, p0z), (c1x, c1y, c1z), f2w, x0,
                     params['fp1'], c_src=128, qblk=512)

    out = _head(f1.reshape(B * N, 128), params['out'])
    return out.reshape(B * N, NUM_CLASSES)


# speedup vs baseline: 4.5023x; 1.0494x over previous
import functools

import jax
import jax.numpy as jnp
import numpy as np
from jax import lax
from jax.experimental import pallas as pl
from jax.experimental.pallas import tpu as pltpu
from jax.experimental.pallas import tpu_sc as plsc

B = 8
N = 4096
F_IN = 3
NUM_CLASSES = 13
RATIOS = (0.2, 0.25)
RADII = (0.2, 0.4)
MAX_NEIGH = 64


S1, S1P = 819, 832
S2, S2P = 205, 256
NP1 = 896
PADPOS = 1e9


def _tile_store(ref, t, val_col):
    Bb = val_col.shape[0]
    tbase = pl.multiple_of((t // 128) * 128, 128)
    lane = jax.lax.broadcasted_iota(jnp.int32, (Bb, 128), 1)
    sel = lane == (t % 128)
    cur = ref[:, pl.ds(tbase, 128)]
    ref[:, pl.ds(tbase, 128)] = jnp.where(sel, jnp.broadcast_to(val_col, (Bb, 128)), cur)


def _fps_kernel(px_ref, py_ref, pz_ref, d0_ref, idx_ref, sx_ref, sy_ref, sz_ref,
                *, n_samples):
    posx = px_ref[...]
    posy = py_ref[...]
    posz = pz_ref[...]
    np_lanes = posx.shape[1]
    Bb = posx.shape[0]
    lane = jax.lax.broadcasted_iota(jnp.int32, posx.shape, 1)
    idx_ref[...] = jnp.zeros(idx_ref.shape, jnp.int32)
    sx_ref[...] = jnp.zeros(sx_ref.shape, jnp.float32)
    sy_ref[...] = jnp.zeros(sy_ref.shape, jnp.float32)
    sz_ref[...] = jnp.zeros(sz_ref.shape, jnp.float32)

    def step(t, carry):
        last, dists = carry
        onehot = lane == last
        px = jnp.sum(jnp.where(onehot, posx, 0.0), axis=1, keepdims=True)
        py = jnp.sum(jnp.where(onehot, posy, 0.0), axis=1, keepdims=True)
        pz = jnp.sum(jnp.where(onehot, posz, 0.0), axis=1, keepdims=True)
        dx = posx - px
        dy = posy - py
        dz = posz - pz
        d2 = (dx * dx + dy * dy) + dz * dz
        dists = jnp.minimum(dists, d2)
        m = jnp.max(dists, axis=1, keepdims=True)
        cand = jnp.where(dists == m, lane, np_lanes)
        nxt = jnp.min(cand, axis=1, keepdims=True)
        _tile_store(idx_ref, t, nxt)
        _tile_store(sx_ref, t - 1, px)
        _tile_store(sy_ref, t - 1, py)
        _tile_store(sz_ref, t - 1, pz)
        return nxt, dists

    last, _ = jax.lax.fori_loop(
        1, n_samples, step,
        (jnp.zeros((Bb, 1), jnp.int32), d0_ref[...]))
    onehot = lane == last
    _tile_store(sx_ref, n_samples - 1,
                jnp.sum(jnp.where(onehot, posx, 0.0), axis=1, keepdims=True))
    _tile_store(sy_ref, n_samples - 1,
                jnp.sum(jnp.where(onehot, posy, 0.0), axis=1, keepdims=True))
    _tile_store(sz_ref, n_samples - 1,
                jnp.sum(jnp.where(onehot, posz, 0.0), axis=1, keepdims=True))


def _fps(posx, posy, posz, n_real, n_samples):
    Bb, Np = posx.shape
    sp = -n_samples % 128 + n_samples
    lane = jax.lax.broadcasted_iota(jnp.int32, (Bb, Np), 1)
    d0 = jnp.where(lane < n_real, jnp.inf, -jnp.inf).astype(jnp.float32)
    out_shapes = (
        jax.ShapeDtypeStruct((Bb, sp), jnp.int32),
        jax.ShapeDtypeStruct((Bb, sp), jnp.float32),
        jax.ShapeDtypeStruct((Bb, sp), jnp.float32),
        jax.ShapeDtypeStruct((Bb, sp), jnp.float32),
    )
    idx, sx, sy, sz = pl.pallas_call(
        functools.partial(_fps_kernel, n_samples=n_samples),
        out_shape=out_shapes,
    )(posx, posy, posz, d0)
    return (idx[:, :n_samples], sx[:, :n_samples], sy[:, :n_samples],
            sz[:, :n_samples])


_SENT = 0x7F000000


def _select_kernel(qx_ref, qy_ref, qz_ref, px_ref, py_ref, pz_ref,
                   keys_ref, *, r2, k):
    qx = qx_ref[0]
    qy = qy_ref[0]
    qz = qz_ref[0]
    px = px_ref[0]
    py = py_ref[0]
    pz = pz_ref[0]
    dx = qx - px
    dy = qy - py
    dz = qz - pz
    d2 = (dx * dx + dy * dy) + dz * dz
    lane = jax.lax.broadcasted_iota(jnp.int32, d2.shape, 1)
    bits = jax.lax.bitcast_convert_type(d2, jnp.int32)
    keys_ref[0] = jnp.where(d2 <= r2, (bits & (~0xFFF)) | lane, _SENT)


def _select(qx, qy, qz, px, py, pz, r2, k, qblk):
    Bb, Qp = qx.shape
    Np = px.shape[1]
    q3 = qx.reshape(Bb, Qp, 1)
    grid = (Bb, Qp // qblk)
    qspec = pl.BlockSpec((1, qblk, 1), lambda b, i: (b, i, 0))
    pspec = pl.BlockSpec((1, 1, Np), lambda b, i: (b, 0, 0))
    return pl.pallas_call(
        functools.partial(_select_kernel, r2=r2, k=k),
        grid=grid,
        in_specs=[qspec, qspec, qspec, pspec, pspec, pspec],
        out_specs=pl.BlockSpec((1, qblk, Np), lambda b, i: (b, i, 0)),
        out_shape=jax.ShapeDtypeStruct((Bb, Qp, Np), jnp.int32),
    )(qx.reshape(Bb, Qp, 1), qy.reshape(Bb, Qp, 1), qz.reshape(Bb, Qp, 1),
      px.reshape(Bb, 1, Np), py.reshape(Bb, 1, Np), pz.reshape(Bb, 1, Np))


def _sa_mlp_kernel(g_ref, qpr_ref, cntr_ref, cnt_ref, w1_ref, w1p_ref, b1_ref,
                   w2_ref, b2_ref, w3_ref, b3_ref, o_ref, *, nneigh):
    g = g_ref[...]
    rows = g.shape[0]
    qb = rows // nneigh
    tq = jnp.dot(qpr_ref[...], w1p_ref[...], preferred_element_type=jnp.float32)
    h = jnp.dot(g, w1_ref[...], preferred_element_type=jnp.float32)
    h = jnp.maximum(h + b1_ref[...][None, :] - tq, 0.0)
    h = jnp.maximum(jnp.dot(h, w2_ref[...], preferred_element_type=jnp.float32)
                    + b2_ref[...][None, :], 0.0)
    h = jnp.dot(h, w3_ref[...], preferred_element_type=jnp.float32) + b3_ref[...][None, :]
    h = jnp.where(cntr_ref[...] > 0, h, -jnp.inf)
    m = jnp.max(h.reshape(qb, nneigh, h.shape[-1]), axis=1)
    o_ref[...] = jnp.where(cnt_ref[...] > 0, m, 0.0)


def _sa_mlp(G, qpos, counts, ps, nneigh, cin_split, qblk, vflags):
    (W1, b1), (W2, b2), (W3, b3) = ps
    Q, Cpad = G.shape[0] // nneigh, G.shape[1]
    W1x = W1[:cin_split]
    W1p = W1[cin_split:cin_split + 3]
    W1pad = jnp.zeros((Cpad, W1.shape[1]), jnp.float32)
    W1pad = W1pad.at[:cin_split].set(W1x).at[cin_split:cin_split + 3].set(W1p)
    qpos_rep = jnp.broadcast_to(qpos[:, None, :], (Q, nneigh, 3)).reshape(Q * nneigh, 3)
    cnt_rep = vflags
    grid = (Q // qblk,)
    return pl.pallas_call(
        functools.partial(_sa_mlp_kernel, nneigh=nneigh),
        grid=grid,
        in_specs=[
            pl.BlockSpec((qblk * nneigh, Cpad), lambda i: (i, 0)),
            pl.BlockSpec((qblk * nneigh, 3), lambda i: (i, 0)),
            pl.BlockSpec((qblk * nneigh, 1), lambda i: (i, 0)),
            pl.BlockSpec((qblk, 1), lambda i: (i, 0)),
            pl.BlockSpec(W1pad.shape, lambda i: (0, 0)),
            pl.BlockSpec(W1p.shape, lambda i: (0, 0)),
            pl.BlockSpec(b1.shape, lambda i: (0,)),
            pl.BlockSpec(W2.shape, lambda i: (0, 0)),
            pl.BlockSpec(b2.shape, lambda i: (0,)),
            pl.BlockSpec(W3.shape, lambda i: (0, 0)),
            pl.BlockSpec(b3.shape, lambda i: (0,)),
        ],
        out_specs=pl.BlockSpec((qblk, W3.shape[1]), lambda i: (i, 0)),
        out_shape=jax.ShapeDtypeStruct((Q, W3.shape[1]), jnp.float32),
    )(G, qpos_rep, cnt_rep, counts, W1pad, W1p, b1, W2, b2, W3, b3)


def _sa3_fp3_kernel(x2_ref, qp_ref, w1_ref, w1p_ref, b1_ref, w2_ref, b2_ref,
                    w3_ref, b3_ref, f1a_ref, f1b_ref, fb1_ref, f2w_ref,
                    fb2_ref, o_ref, *, n_real):
    x2 = x2_ref[0]
    qp = qp_ref[0]
    h = jnp.dot(x2, w1_ref[...], preferred_element_type=jnp.float32)
    h = h + jnp.dot(qp, w1p_ref[...], preferred_element_type=jnp.float32)
    h = jnp.maximum(h + b1_ref[...][None, :], 0.0)
    h = jnp.maximum(jnp.dot(h, w2_ref[...], preferred_element_type=jnp.float32)
                    + b2_ref[...][None, :], 0.0)
    h = jnp.dot(h, w3_ref[...], preferred_element_type=jnp.float32) + b3_ref[...][None, :]
    row = jax.lax.broadcasted_iota(jnp.int32, h.shape, 0)
    h = jnp.where(row < n_real, h, -jnp.inf)
    xg = jnp.max(h, axis=0, keepdims=True)
    f = jnp.dot(x2, f1b_ref[...], preferred_element_type=jnp.float32)
    f = f + jnp.dot(xg, f1a_ref[...], preferred_element_type=jnp.float32)
    f = jnp.maximum(f + fb1_ref[...][None, :], 0.0)
    f = jnp.dot(f, f2w_ref[...], preferred_element_type=jnp.float32) + fb2_ref[...][None, :]
    row2 = jax.lax.broadcasted_iota(jnp.int32, f.shape, 0)
    o_ref[0] = jnp.where(row2 < n_real, f, 0.0)


def _sa3_fp3(x2, pos2, ps3, psf, n_real):
    (W1, b1), (W2, b2), (W3, b3) = ps3
    (F1, fb1), (F2, fb2) = psf
    Bb, S2p, C = x2.shape
    W1x = W1[:C]
    W1p = W1[C:C + 3]
    F1a = F1[:W3.shape[1]]
    F1b = F1[W3.shape[1]:]
    grid = (Bb,)
    return pl.pallas_call(
        functools.partial(_sa3_fp3_kernel, n_real=n_real),
        grid=grid,
        in_specs=[
            pl.BlockSpec((1, S2p, C), lambda b: (b, 0, 0)),
            pl.BlockSpec((1, S2p, 3), lambda b: (b, 0, 0)),
            pl.BlockSpec(W1x.shape, lambda b: (0, 0)),
            pl.BlockSpec(W1p.shape, lambda b: (0, 0)),
            pl.BlockSpec(b1.shape, lambda b: (0,)),
            pl.BlockSpec(W2.shape, lambda b: (0, 0)),
            pl.BlockSpec(b2.shape, lambda b: (0,)),
            pl.BlockSpec(W3.shape, lambda b: (0, 0)),
            pl.BlockSpec(b3.shape, lambda b: (0,)),
            pl.BlockSpec(F1a.shape, lambda b: (0, 0)),
            pl.BlockSpec(F1b.shape, lambda b: (0, 0)),
            pl.BlockSpec(fb1.shape, lambda b: (0,)),
            pl.BlockSpec(F2.shape, lambda b: (0, 0)),
            pl.BlockSpec(fb2.shape, lambda b: (0,)),
        ],
        out_specs=pl.BlockSpec((1, S2p, F2.shape[1]), lambda b: (b, 0, 0)),
        out_shape=jax.ShapeDtypeStruct((Bb, S2p, F2.shape[1]), jnp.float32),
    )(x2, pos2, W1x, W1p, b1, W2, b2, W3, b3, F1a, F1b, fb1, F2, fb2)


def _interp_kernel(qx_ref, qy_ref, qz_ref, px_ref, py_ref, pz_ref, f_ref,
                   xskip_ref, wa_ref, wb_ref, b1_ref, *rest, n_layers):
    if n_layers == 2:
        w2_ref, b2_ref, o_ref = rest
    else:
        w2_ref, b2_ref, w3_ref, b3_ref, o_ref = rest
    qx = qx_ref[0]
    qy = qy_ref[0]
    qz = qz_ref[0]
    px = px_ref[0]
    py = py_ref[0]
    pz = pz_ref[0]
    dx = qx - px
    dy = qy - py
    dz = qz - pz
    d2 = (dx * dx + dy * dy) + dz * dz
    lane = jax.lax.broadcasted_iota(jnp.int32, d2.shape, 1)
    qb, ncand = d2.shape
    Wint = jnp.zeros(d2.shape, jnp.float32)
    wsum = jnp.zeros((qb, 1), jnp.float32)
    for _ in range(3):
        m = jnp.min(d2, axis=1, keepdims=True)
        cand = jnp.where(d2 == m, lane, ncand)
        idxk = jnp.min(cand, axis=1, keepdims=True)
        w = 1.0 / jnp.maximum(m, 1e-16)
        hit = lane == idxk
        Wint = jnp.where(hit, w, Wint)
        wsum = wsum + w
        d2 = jnp.where(hit, jnp.inf, d2)
    Wint = Wint / wsum
    xi = jnp.dot(Wint, f_ref[0], preferred_element_type=jnp.float32)
    h = jnp.dot(xi, wa_ref[...], preferred_element_type=jnp.float32)
    h = h + jnp.dot(xskip_ref[0], wb_ref[...], preferred_element_type=jnp.float32)
    h = jnp.maximum(h + b1_ref[...][None, :], 0.0)
    h = jnp.dot(h, w2_ref[...], preferred_element_type=jnp.float32) + b2_ref[...][None, :]
    if n_layers == 3:
        h = jnp.maximum(h, 0.0)
        h = jnp.dot(h, w3_ref[...], preferred_element_type=jnp.float32) + b3_ref[...][None, :]
    o_ref[0] = h


def _interp_mlp(qplanes, cplanes, f_src, x_skip, ps, c_src, qblk):
    (W1, b1), *restps = ps
    Bb, Qp = qplanes[0].shape
    Ncp = cplanes[0].shape[1]
    Cs = x_skip.shape[-1]
    Wa = W1[:c_src]
    Wb = W1[c_src:]
    n_layers = 1 + len(restps)
    grid = (Bb, Qp // qblk)
    qspec = pl.BlockSpec((1, qblk, 1), lambda b, i: (b, i, 0))
    cspec = pl.BlockSpec((1, 1, Ncp), lambda b, i: (b, 0, 0))
    ins = [q.reshape(Bb, Qp, 1) for q in qplanes] + \
          [c.reshape(Bb, 1, Ncp) for c in cplanes] + [f_src, x_skip]
    in_specs = [qspec] * 3 + [cspec] * 3 + [
        pl.BlockSpec((1, Ncp, c_src), lambda b, i: (b, 0, 0)),
        pl.BlockSpec((1, qblk, Cs), lambda b, i: (b, i, 0)),
        pl.BlockSpec(Wa.shape, lambda b, i: (0, 0)),
        pl.BlockSpec(Wb.shape, lambda b, i: (0, 0)),
        pl.BlockSpec(b1.shape, lambda b, i: (0,)),
    ]
    args = ins + [Wa, Wb, b1]
    for (W, b) in restps:
        in_specs += [pl.BlockSpec(W.shape, lambda b, i: (0, 0)),
                     pl.BlockSpec(b.shape, lambda b, i: (0,))]
        args += [W, b]
    Cout = restps[-1][0].shape[1]
    return pl.pallas_call(
        functools.partial(_interp_kernel, n_layers=n_layers),
        grid=grid,
        in_specs=in_specs,
        out_specs=pl.BlockSpec((1, qblk, Cout), lambda b, i: (b, i, 0)),
        out_shape=jax.ShapeDtypeStruct((Bb, Qp, Cout), jnp.float32),
    )(*args)


def _head_kernel(f_ref, w0, b0, w1, b1, w2, b2, o_ref):
    h = f_ref[...]
    h = jnp.maximum(h @ w0[...] + b0[...][None, :], 0.0)
    h = jnp.maximum(h @ w1[...] + b1[...][None, :], 0.0)
    o = h @ w2[...] + b2[...][None, :]
    o = o - jax.scipy.special.logsumexp(o, axis=-1, keepdims=True)
    o_ref[...] = o


def _head(f1, ps):
    (w0, b0), (w1, b1), (w2, b2) = ps
    M = f1.shape[0]
    TILE = 2048
    grid = (M // TILE,)
    return pl.pallas_call(
        _head_kernel,
        grid=grid,
        in_specs=[
            pl.BlockSpec((TILE, f1.shape[1]), lambda i: (i, 0)),
            pl.BlockSpec(w0.shape, lambda i: (0, 0)),
            pl.BlockSpec(b0.shape, lambda i: (0,)),
            pl.BlockSpec(w1.shape, lambda i: (0, 0)),
            pl.BlockSpec(b1.shape, lambda i: (0,)),
            pl.BlockSpec(w2.shape, lambda i: (0, 0)),
            pl.BlockSpec(b2.shape, lambda i: (0,)),
        ],
        out_specs=pl.BlockSpec((TILE, NUM_CLASSES), lambda i: (i, 0)),
        out_shape=jax.ShapeDtypeStruct((M, NUM_CLASSES), jnp.float32),
    )(f1, w0, b0, w1, b1, w2, b2)


_SC_TILES = 32


def _sc_compact_gather(keys, tau, table, rows_per_b, cand_pb):
    R, Np = keys.shape
    T, C = table.shape
    NR = R // _SC_TILES
    n_chunks = Np // 16
    tau16 = jnp.broadcast_to(tau[:, None], (R, 16))
    mesh = plsc.VectorSubcoreMesh(core_axis_name="c", subcore_axis_name="s")

    @functools.partial(
        pl.kernel, mesh=mesh,
        out_type=[
            jax.ShapeDtypeStruct((R * MAX_NEIGH, C), jnp.float32),
            jax.ShapeDtypeStruct((R * 16,), jnp.int32),
        ],
        scratch_types=[
            pltpu.VMEM((16,), jnp.int32),
            pltpu.VMEM((Np,), jnp.int32),
            pltpu.VMEM((MAX_NEIGH,), jnp.int32),
            pltpu.VMEM((MAX_NEIGH, C), jnp.float32),
            pltpu.VMEM((NR * 16,), jnp.int32),
            pltpu.SemaphoreType.DMA,
        ],
    )
    def k(keys_hbm, tau_hbm, table_hbm, g_out, cnt_out,
          tau_v, kbuf, idxbuf, gbuf, cnt_v, sem):
        wid = lax.axis_index("s") * 2 + lax.axis_index("c")
        base = wid * NR
        iota = lax.iota(jnp.int32, 16)
        for c4 in range(MAX_NEIGH // 16):
            idxbuf[pl.ds(c4 * 16, 16)] = jnp.zeros((16,), jnp.int32)

        def row_body(rloc, _):
            r = base + rloc
            gbase = (r // rows_per_b) * cand_pb
            pltpu.sync_copy(keys_hbm.at[r], kbuf)
            pltpu.sync_copy(tau_hbm.at[r], tau_v)
            t16 = tau_v[...]

            def chunk(c, off):
                kk = kbuf[pl.ds(c * 16, 16)]
                m = kk <= t16
                mi = m.astype(jnp.int32)
                cum = plsc.cumsum(mi)
                pos = cum + (off - 1)
                vals = iota + (gbase + c * 16)
                plsc.store_scatter(idxbuf, [pos], vals, mask=m)
                return off + jnp.sum(mi)

            off = lax.fori_loop(0, n_chunks, chunk, jnp.int32(0))
            cnt_v[pl.ds(rloc * 16, 16)] = jnp.zeros((16,), jnp.int32) + off
            pltpu.async_copy(table_hbm.at[idxbuf], gbuf, sem).wait()
            pltpu.sync_copy(gbuf, g_out.at[pl.ds(r * MAX_NEIGH, MAX_NEIGH)])
            return 0

        lax.fori_loop(0, NR, row_body, 0)
        pltpu.sync_copy(cnt_v, cnt_out.at[pl.ds(base * 16, NR * 16)])

    G, cnt16 = k(keys, tau16, table)
    return G, cnt16.reshape(R, 16)[:, 0]


def _tie(dep, *xs):
    out = lax.optimization_barrier((dep, *xs))
    return out[1:] if len(xs) > 1 else out[1]


def _plane_pad(p, npad, fill=PADPOS):
    return jnp.pad(p, ((0, 0), (0, npad - p.shape[1])), constant_values=fill)


def _neighbors(keys):
    _, nidx = jax.lax.approx_min_k(keys.astype(jnp.float32), MAX_NEIGH,
                                   recall_target=0.99)
    nidx = nidx.astype(jnp.int32)
    kg = jnp.take_along_axis(keys, nidx, axis=-1)
    valid = (kg < _SENT).astype(jnp.int32)
    cnt = jnp.sum(valid, axis=-1)
    nidx = jnp.where(valid > 0, nidx, 0)
    return nidx, valid, cnt


def kernel(x, pos, batch, params):
    del batch
    x0 = x.reshape(B, N, F_IN)
    p0 = pos.reshape(B, N, 3)
    p0x, p0y, p0z = p0[..., 0], p0[..., 1], p0[..., 2]

    _, s1x, s1y, s1z = _fps(p0x, p0y, p0z, N, S1)
    q1x, q1y, q1z = (_plane_pad(s, S1P) for s in (s1x, s1y, s1z))
    keys1 = _select(q1x, q1y, q1z, p0x, p0y, p0z, RADII[0] ** 2,
                    MAX_NEIGH, qblk=208)
    table1 = jnp.concatenate(
        [x0, p0, jnp.zeros((B, N, 10), jnp.float32)], axis=-1).reshape(B * N, 16)
    nidx1, val1, cnt1 = _neighbors(keys1)
    gid1 = (jnp.arange(B, dtype=jnp.int32)[:, None, None] * N + nidx1).reshape(-1)
    G1 = table1[gid1]
    qpos1 = jnp.stack([q1x, q1y, q1z], axis=-1).reshape(B * S1P, 3)
    x1 = _sa_mlp(G1, qpos1, cnt1.reshape(-1, 1), params['sa1'], MAX_NEIGH,
                 cin_split=3, qblk=104, vflags=val1.reshape(-1, 1))
    x1 = x1.reshape(B, S1P, 128)

    c1x, c1y, c1z = (_plane_pad(s, NP1) for s in (s1x, s1y, s1z))
    c1x, c1y, c1z = _tie(cnt1, c1x, c1y, c1z)
    _, s2x, s2y, s2z = _fps(c1x, c1y, c1z, S1, S2)
    q2x, q2y, q2z = (_plane_pad(s, S2P) for s in (s2x, s2y, s2z))
    keys2 = _select(q2x, q2y, q2z, c1x, c1y, c1z, RADII[1] ** 2,
                    MAX_NEIGH, qblk=256)
    x1w = jnp.pad(x1, ((0, 0), (0, NP1 - S1P), (0, 0)))
    table2 = jnp.concatenate(
        [x1w, jnp.stack([c1x, c1y, c1z], axis=-1),
         jnp.zeros((B, NP1, 13), jnp.float32)], axis=-1).reshape(B * NP1, 144)
    nidx2, val2, cnt2 = _neighbors(keys2)
    gid2 = (jnp.arange(B, dtype=jnp.int32)[:, None, None] * NP1 + nidx2).reshape(-1)
    G2 = table2[gid2]
    qpos2 = jnp.stack([q2x, q2y, q2z], axis=-1).reshape(B * S2P, 3)
    x2 = _sa_mlp(G2, qpos2, cnt2.reshape(-1, 1), params['sa2'], MAX_NEIGH,
                 cin_split=128, qblk=64, vflags=val2.reshape(-1, 1))
    x2 = x2.reshape(B, S2P, 256)

    x2 = _tie(cnt2, x2)
    pos2 = jnp.stack([q2x, q2y, q2z], axis=-1)
    f3 = _sa3_fp3(x2, pos2, params['sa3'], params['fp3'], S2)

    f2 = _interp_mlp((q1x, q1y, q1z), (q2x, q2y, q2z), f3, x1,
                     params['fp2'], c_src=256, qblk=208)

    f2w = jnp.pad(f2, ((0, 0), (0, NP1 - S1P), (0, 0)))
    f1 = _interp_mlp((p0x, p0y, p0z), (c1x, c1y, c1z), f2w, x0,
                     params['fp1'], c_src=128, qblk=512)

    out = _head(f1.reshape(B * N, 128), params['out'])
    return out.reshape(B * N, NUM_CLASSES)


# speedup vs baseline: 4.5471x; 1.0100x over previous
import functools

import jax
import jax.numpy as jnp
import numpy as np
from jax import lax
from jax.experimental import pallas as pl
from jax.experimental.pallas import tpu as pltpu
from jax.experimental.pallas import tpu_sc as plsc

B = 8
N = 4096
F_IN = 3
NUM_CLASSES = 13
RATIOS = (0.2, 0.25)
RADII = (0.2, 0.4)
MAX_NEIGH = 64


S1, S1P = 819, 832
S2, S2P = 205, 256
NP1 = 896
PADPOS = 1e9


def _tile_store(ref, t, val_col):
    Bb = val_col.shape[0]
    tbase = pl.multiple_of((t // 128) * 128, 128)
    lane = jax.lax.broadcasted_iota(jnp.int32, (Bb, 128), 1)
    sel = lane == (t % 128)
    cur = ref[:, pl.ds(tbase, 128)]
    ref[:, pl.ds(tbase, 128)] = jnp.where(sel, jnp.broadcast_to(val_col, (Bb, 128)), cur)


def _fps_kernel(px_ref, py_ref, pz_ref, d0_ref, idx_ref, sx_ref, sy_ref, sz_ref,
                *, n_samples):
    posx = px_ref[...]
    posy = py_ref[...]
    posz = pz_ref[...]
    np_lanes = posx.shape[1]
    Bb = posx.shape[0]
    lane = jax.lax.broadcasted_iota(jnp.int32, posx.shape, 1)
    idx_ref[...] = jnp.zeros(idx_ref.shape, jnp.int32)
    sx_ref[...] = jnp.zeros(sx_ref.shape, jnp.float32)
    sy_ref[...] = jnp.zeros(sy_ref.shape, jnp.float32)
    sz_ref[...] = jnp.zeros(sz_ref.shape, jnp.float32)

    def step(t, carry):
        last, dists = carry
        onehot = lane == last
        px = jnp.sum(jnp.where(onehot, posx, 0.0), axis=1, keepdims=True)
        py = jnp.sum(jnp.where(onehot, posy, 0.0), axis=1, keepdims=True)
        pz = jnp.sum(jnp.where(onehot, posz, 0.0), axis=1, keepdims=True)
        dx = posx - px
        dy = posy - py
        dz = posz - pz
        d2 = (dx * dx + dy * dy) + dz * dz
        dists = jnp.minimum(dists, d2)
        m = jnp.max(dists, axis=1, keepdims=True)
        cand = jnp.where(dists == m, lane, np_lanes)
        nxt = jnp.min(cand, axis=1, keepdims=True)
        _tile_store(idx_ref, t, nxt)
        _tile_store(sx_ref, t - 1, px)
        _tile_store(sy_ref, t - 1, py)
        _tile_store(sz_ref, t - 1, pz)
        return nxt, dists

    last, _ = jax.lax.fori_loop(
        1, n_samples, step,
        (jnp.zeros((Bb, 1), jnp.int32), d0_ref[...]))
    onehot = lane == last
    _tile_store(sx_ref, n_samples - 1,
                jnp.sum(jnp.where(onehot, posx, 0.0), axis=1, keepdims=True))
    _tile_store(sy_ref, n_samples - 1,
                jnp.sum(jnp.where(onehot, posy, 0.0), axis=1, keepdims=True))
    _tile_store(sz_ref, n_samples - 1,
                jnp.sum(jnp.where(onehot, posz, 0.0), axis=1, keepdims=True))


def _fps(posx, posy, posz, n_real, n_samples):
    Bb, Np = posx.shape
    sp = -n_samples % 128 + n_samples
    lane = jax.lax.broadcasted_iota(jnp.int32, (Bb, Np), 1)
    d0 = jnp.where(lane < n_real, jnp.inf, -jnp.inf).astype(jnp.float32)
    out_shapes = (
        jax.ShapeDtypeStruct((Bb, sp), jnp.int32),
        jax.ShapeDtypeStruct((Bb, sp), jnp.float32),
        jax.ShapeDtypeStruct((Bb, sp), jnp.float32),
        jax.ShapeDtypeStruct((Bb, sp), jnp.float32),
    )
    idx, sx, sy, sz = pl.pallas_call(
        functools.partial(_fps_kernel, n_samples=n_samples),
        out_shape=out_shapes,
    )(posx, posy, posz, d0)
    return (idx[:, :n_samples], sx[:, :n_samples], sy[:, :n_samples],
            sz[:, :n_samples])


_SENT = 0x7F000000


def _select_kernel(qx_ref, qy_ref, qz_ref, px_ref, py_ref, pz_ref,
                   keys_ref, *, r2, k):
    qx = qx_ref[0]
    qy = qy_ref[0]
    qz = qz_ref[0]
    px = px_ref[0]
    py = py_ref[0]
    pz = pz_ref[0]
    dx = qx - px
    dy = qy - py
    dz = qz - pz
    d2 = (dx * dx + dy * dy) + dz * dz
    lane = jax.lax.broadcasted_iota(jnp.int32, d2.shape, 1)
    bits = jax.lax.bitcast_convert_type(d2, jnp.int32)
    keys = jnp.where(d2 <= r2, (bits & (~0xFFF)) | lane, _SENT)
    keys_ref[0] = keys.astype(jnp.float32)


def _select(qx, qy, qz, px, py, pz, r2, k, qblk):
    Bb, Qp = qx.shape
    Np = px.shape[1]
    q3 = qx.reshape(Bb, Qp, 1)
    grid = (Bb, Qp // qblk)
    qspec = pl.BlockSpec((1, qblk, 1), lambda b, i: (b, i, 0))
    pspec = pl.BlockSpec((1, 1, Np), lambda b, i: (b, 0, 0))
    return pl.pallas_call(
        functools.partial(_select_kernel, r2=r2, k=k),
        grid=grid,
        in_specs=[qspec, qspec, qspec, pspec, pspec, pspec],
        out_specs=pl.BlockSpec((1, qblk, Np), lambda b, i: (b, i, 0)),
        out_shape=jax.ShapeDtypeStruct((Bb, Qp, Np), jnp.float32),
    )(qx.reshape(Bb, Qp, 1), qy.reshape(Bb, Qp, 1), qz.reshape(Bb, Qp, 1),
      px.reshape(Bb, 1, Np), py.reshape(Bb, 1, Np), pz.reshape(Bb, 1, Np))


def _sa_mlp_kernel(g_ref, qpr_ref, cntr_ref, cnt_ref, w1_ref, w1p_ref, b1_ref,
                   w2_ref, b2_ref, w3_ref, b3_ref, o_ref, *, nneigh):
    g = g_ref[...]
    rows = g.shape[0]
    qb = rows // nneigh
    tq = jnp.dot(qpr_ref[...], w1p_ref[...], preferred_element_type=jnp.float32)
    h = jnp.dot(g, w1_ref[...], preferred_element_type=jnp.float32)
    h = jnp.maximum(h + b1_ref[...][None, :] - tq, 0.0)
    h = jnp.maximum(jnp.dot(h, w2_ref[...], preferred_element_type=jnp.float32)
                    + b2_ref[...][None, :], 0.0)
    h = jnp.dot(h, w3_ref[...], preferred_element_type=jnp.float32) + b3_ref[...][None, :]
    h = jnp.where(cntr_ref[...] > 0, h, -jnp.inf)
    m = jnp.max(h.reshape(qb, nneigh, h.shape[-1]), axis=1)
    o_ref[...] = jnp.where(cnt_ref[...] > 0, m, 0.0)


def _sa_mlp(G, qpos, counts, ps, nneigh, cin_split, qblk, vflags):
    (W1, b1), (W2, b2), (W3, b3) = ps
    Q, Cpad = G.shape[0] // nneigh, G.shape[1]
    W1x = W1[:cin_split]
    W1p = W1[cin_split:cin_split + 3]
    W1pad = jnp.zeros((Cpad, W1.shape[1]), jnp.float32)
    W1pad = W1pad.at[:cin_split].set(W1x).at[cin_split:cin_split + 3].set(W1p)
    qpos_rep = jnp.broadcast_to(qpos[:, None, :], (Q, nneigh, 3)).reshape(Q * nneigh, 3)
    cnt_rep = vflags
    grid = (Q // qblk,)
    return pl.pallas_call(
        functools.partial(_sa_mlp_kernel, nneigh=nneigh),
        grid=grid,
        in_specs=[
            pl.BlockSpec((qblk * nneigh, Cpad), lambda i: (i, 0)),
            pl.BlockSpec((qblk * nneigh, 3), lambda i: (i, 0)),
            pl.BlockSpec((qblk * nneigh, 1), lambda i: (i, 0)),
            pl.BlockSpec((qblk, 1), lambda i: (i, 0)),
            pl.BlockSpec(W1pad.shape, lambda i: (0, 0)),
            pl.BlockSpec(W1p.shape, lambda i: (0, 0)),
            pl.BlockSpec(b1.shape, lambda i: (0,)),
            pl.BlockSpec(W2.shape, lambda i: (0, 0)),
            pl.BlockSpec(b2.shape, lambda i: (0,)),
            pl.BlockSpec(W3.shape, lambda i: (0, 0)),
            pl.BlockSpec(b3.shape, lambda i: (0,)),
        ],
        out_specs=pl.BlockSpec((qblk, W3.shape[1]), lambda i: (i, 0)),
        out_shape=jax.ShapeDtypeStruct((Q, W3.shape[1]), jnp.float32),
    )(G, qpos_rep, cnt_rep, counts, W1pad, W1p, b1, W2, b2, W3, b3)


def _sa3_fp3_kernel(x2_ref, qp_ref, w1_ref, w1p_ref, b1_ref, w2_ref, b2_ref,
                    w3_ref, b3_ref, f1a_ref, f1b_ref, fb1_ref, f2w_ref,
                    fb2_ref, o_ref, *, n_real):
    x2 = x2_ref[0]
    qp = qp_ref[0]
    h = jnp.dot(x2, w1_ref[...], preferred_element_type=jnp.float32)
    h = h + jnp.dot(qp, w1p_ref[...], preferred_element_type=jnp.float32)
    h = jnp.maximum(h + b1_ref[...][None, :], 0.0)
    h = jnp.maximum(jnp.dot(h, w2_ref[...], preferred_element_type=jnp.float32)
                    + b2_ref[...][None, :], 0.0)
    h = jnp.dot(h, w3_ref[...], preferred_element_type=jnp.float32) + b3_ref[...][None, :]
    row = jax.lax.broadcasted_iota(jnp.int32, h.shape, 0)
    h = jnp.where(row < n_real, h, -jnp.inf)
    xg = jnp.max(h, axis=0, keepdims=True)
    f = jnp.dot(x2, f1b_ref[...], preferred_element_type=jnp.float32)
    f = f + jnp.dot(xg, f1a_ref[...], preferred_element_type=jnp.float32)
    f = jnp.maximum(f + fb1_ref[...][None, :], 0.0)
    f = jnp.dot(f, f2w_ref[...], preferred_element_type=jnp.float32) + fb2_ref[...][None, :]
    row2 = jax.lax.broadcasted_iota(jnp.int32, f.shape, 0)
    o_ref[0] = jnp.where(row2 < n_real, f, 0.0)


def _sa3_fp3(x2, pos2, ps3, psf, n_real):
    (W1, b1), (W2, b2), (W3, b3) = ps3
    (F1, fb1), (F2, fb2) = psf
    Bb, S2p, C = x2.shape
    W1x = W1[:C]
    W1p = W1[C:C + 3]
    F1a = F1[:W3.shape[1]]
    F1b = F1[W3.shape[1]:]
    grid = (Bb,)
    return pl.pallas_call(
        functools.partial(_sa3_fp3_kernel, n_real=n_real),
        grid=grid,
        in_specs=[
            pl.BlockSpec((1, S2p, C), lambda b: (b, 0, 0)),
            pl.BlockSpec((1, S2p, 3), lambda b: (b, 0, 0)),
            pl.BlockSpec(W1x.shape, lambda b: (0, 0)),
            pl.BlockSpec(W1p.shape, lambda b: (0, 0)),
            pl.BlockSpec(b1.shape, lambda b: (0,)),
            pl.BlockSpec(W2.shape, lambda b: (0, 0)),
            pl.BlockSpec(b2.shape, lambda b: (0,)),
            pl.BlockSpec(W3.shape, lambda b: (0, 0)),
            pl.BlockSpec(b3.shape, lambda b: (0,)),
            pl.BlockSpec(F1a.shape, lambda b: (0, 0)),
            pl.BlockSpec(F1b.shape, lambda b: (0, 0)),
            pl.BlockSpec(fb1.shape, lambda b: (0,)),
            pl.BlockSpec(F2.shape, lambda b: (0, 0)),
            pl.BlockSpec(fb2.shape, lambda b: (0,)),
        ],
        out_specs=pl.BlockSpec((1, S2p, F2.shape[1]), lambda b: (b, 0, 0)),
        out_shape=jax.ShapeDtypeStruct((Bb, S2p, F2.shape[1]), jnp.float32),
    )(x2, pos2, W1x, W1p, b1, W2, b2, W3, b3, F1a, F1b, fb1, F2, fb2)


def _interp_kernel(qx_ref, qy_ref, qz_ref, px_ref, py_ref, pz_ref, f_ref,
                   xskip_ref, wa_ref, wb_ref, b1_ref, *rest, n_layers):
    if n_layers == 2:
        w2_ref, b2_ref, o_ref = rest
    else:
        w2_ref, b2_ref, w3_ref, b3_ref, o_ref = rest
    qx = qx_ref[0]
    qy = qy_ref[0]
    qz = qz_ref[0]
    px = px_ref[0]
    py = py_ref[0]
    pz = pz_ref[0]
    dx = qx - px
    dy = qy - py
    dz = qz - pz
    d2 = (dx * dx + dy * dy) + dz * dz
    lane = jax.lax.broadcasted_iota(jnp.int32, d2.shape, 1)
    qb, ncand = d2.shape
    Wint = jnp.zeros(d2.shape, jnp.float32)
    wsum = jnp.zeros((qb, 1), jnp.float32)
    for _ in range(3):
        m = jnp.min(d2, axis=1, keepdims=True)
        cand = jnp.where(d2 == m, lane, ncand)
        idxk = jnp.min(cand, axis=1, keepdims=True)
        w = 1.0 / jnp.maximum(m, 1e-16)
        hit = lane == idxk
        Wint = jnp.where(hit, w, Wint)
        wsum = wsum + w
        d2 = jnp.where(hit, jnp.inf, d2)
    Wint = Wint / wsum
    xi = jnp.dot(Wint, f_ref[0], preferred_element_type=jnp.float32)
    h = jnp.dot(xi, wa_ref[...], preferred_element_type=jnp.float32)
    h = h + jnp.dot(xskip_ref[0], wb_ref[...], preferred_element_type=jnp.float32)
    h = jnp.maximum(h + b1_ref[...][None, :], 0.0)
    h = jnp.dot(h, w2_ref[...], preferred_element_type=jnp.float32) + b2_ref[...][None, :]
    if n_layers == 3:
        h = jnp.maximum(h, 0.0)
        h = jnp.dot(h, w3_ref[...], preferred_element_type=jnp.float32) + b3_ref[...][None, :]
    o_ref[0] = h


def _interp_mlp(qplanes, cplanes, f_src, x_skip, ps, c_src, qblk):
    (W1, b1), *restps = ps
    Bb, Qp = qplanes[0].shape
    Ncp = cplanes[0].shape[1]
    Cs = x_skip.shape[-1]
    Wa = W1[:c_src]
    Wb = W1[c_src:]
    n_layers = 1 + len(restps)
    grid = (Bb, Qp // qblk)
    qspec = pl.BlockSpec((1, qblk, 1), lambda b, i: (b, i, 0))
    cspec = pl.BlockSpec((1, 1, Ncp), lambda b, i: (b, 0, 0))
    ins = [q.reshape(Bb, Qp, 1) for q in qplanes] + \
          [c.reshape(Bb, 1, Ncp) for c in cplanes] + [f_src, x_skip]
    in_specs = [qspec] * 3 + [cspec] * 3 + [
        pl.BlockSpec((1, Ncp, c_src), lambda b, i: (b, 0, 0)),
        pl.BlockSpec((1, qblk, Cs), lambda b, i: (b, i, 0)),
        pl.BlockSpec(Wa.shape, lambda b, i: (0, 0)),
        pl.BlockSpec(Wb.shape, lambda b, i: (0, 0)),
        pl.BlockSpec(b1.shape, lambda b, i: (0,)),
    ]
    args = ins + [Wa, Wb, b1]
    for (W, b) in restps:
        in_specs += [pl.BlockSpec(W.shape, lambda b, i: (0, 0)),
                     pl.BlockSpec(b.shape, lambda b, i: (0,))]
        args += [W, b]
    Cout = restps[-1][0].shape[1]
    return pl.pallas_call(
        functools.partial(_interp_kernel, n_layers=n_layers),
        grid=grid,
        in_specs=in_specs,
        out_specs=pl.BlockSpec((1, qblk, Cout), lambda b, i: (b, i, 0)),
        out_shape=jax.ShapeDtypeStruct((Bb, Qp, Cout), jnp.float32),
    )(*args)


def _head_kernel(f_ref, w0, b0, w1, b1, w2, b2, o_ref):
    h = f_ref[...]
    h = jnp.maximum(h @ w0[...] + b0[...][None, :], 0.0)
    h = jnp.maximum(h @ w1[...] + b1[...][None, :], 0.0)
    o = h @ w2[...] + b2[...][None, :]
    o = o - jax.scipy.special.logsumexp(o, axis=-1, keepdims=True)
    o_ref[...] = o


def _head(f1, ps):
    (w0, b0), (w1, b1), (w2, b2) = ps
    M = f1.shape[0]
    TILE = 2048
    grid = (M // TILE,)
    return pl.pallas_call(
        _head_kernel,
        grid=grid,
        in_specs=[
            pl.BlockSpec((TILE, f1.shape[1]), lambda i: (i, 0)),
            pl.BlockSpec(w0.shape, lambda i: (0, 0)),
            pl.BlockSpec(b0.shape, lambda i: (0,)),
            pl.BlockSpec(w1.shape, lambda i: (0, 0)),
            pl.BlockSpec(b1.shape, lambda i: (0,)),
            pl.BlockSpec(w2.shape, lambda i: (0, 0)),
            pl.BlockSpec(b2.shape, lambda i: (0,)),
        ],
        out_specs=pl.BlockSpec((TILE, NUM_CLASSES), lambda i: (i, 0)),
        out_shape=jax.ShapeDtypeStruct((M, NUM_CLASSES), jnp.float32),
    )(f1, w0, b0, w1, b1, w2, b2)


_SC_TILES = 32


def _sc_compact_gather(keys, tau, table, rows_per_b, cand_pb):
    R, Np = keys.shape
    T, C = table.shape
    NR = R // _SC_TILES
    n_chunks = Np // 16
    tau16 = jnp.broadcast_to(tau[:, None], (R, 16))
    mesh = plsc.VectorSubcoreMesh(core_axis_name="c", subcore_axis_name="s")

    @functools.partial(
        pl.kernel, mesh=mesh,
        out_type=[
            jax.ShapeDtypeStruct((R * MAX_NEIGH, C), jnp.float32),
            jax.ShapeDtypeStruct((R * 16,), jnp.int32),
        ],
        scratch_types=[
            pltpu.VMEM((16,), jnp.int32),
            pltpu.VMEM((Np,), jnp.int32),
            pltpu.VMEM((MAX_NEIGH,), jnp.int32),
            pltpu.VMEM((MAX_NEIGH, C), jnp.float32),
            pltpu.VMEM((NR * 16,), jnp.int32),
            pltpu.SemaphoreType.DMA,
        ],
    )
    def k(keys_hbm, tau_hbm, table_hbm, g_out, cnt_out,
          tau_v, kbuf, idxbuf, gbuf, cnt_v, sem):
        wid = lax.axis_index("s") * 2 + lax.axis_index("c")
        base = wid * NR
        iota = lax.iota(jnp.int32, 16)
        for c4 in range(MAX_NEIGH // 16):
            idxbuf[pl.ds(c4 * 16, 16)] = jnp.zeros((16,), jnp.int32)

        def row_body(rloc, _):
            r = base + rloc
            gbase = (r // rows_per_b) * cand_pb
            pltpu.sync_copy(keys_hbm.at[r], kbuf)
            pltpu.sync_copy(tau_hbm.at[r], tau_v)
            t16 = tau_v[...]

            def chunk(c, off):
                kk = kbuf[pl.ds(c * 16, 16)]
                m = kk <= t16
                mi = m.astype(jnp.int32)
                cum = plsc.cumsum(mi)
                pos = cum + (off - 1)
                vals = iota + (gbase + c * 16)
                plsc.store_scatter(idxbuf, [pos], vals, mask=m)
                return off + jnp.sum(mi)

            off = lax.fori_loop(0, n_chunks, chunk, jnp.int32(0))
            cnt_v[pl.ds(rloc * 16, 16)] = jnp.zeros((16,), jnp.int32) + off
            pltpu.async_copy(table_hbm.at[idxbuf], gbuf, sem).wait()
            pltpu.sync_copy(gbuf, g_out.at[pl.ds(r * MAX_NEIGH, MAX_NEIGH)])
            return 0

        lax.fori_loop(0, NR, row_body, 0)
        pltpu.sync_copy(cnt_v, cnt_out.at[pl.ds(base * 16, NR * 16)])

    G, cnt16 = k(keys, tau16, table)
    return G, cnt16.reshape(R, 16)[:, 0]


def _tie(dep, *xs):
    out = lax.optimization_barrier((dep, *xs))
    return out[1:] if len(xs) > 1 else out[1]


def _plane_pad(p, npad, fill=PADPOS):
    return jnp.pad(p, ((0, 0), (0, npad - p.shape[1])), constant_values=fill)


def _neighbors(keys):
    vals, nidx = jax.lax.approx_min_k(keys, MAX_NEIGH, recall_target=0.99)
    nidx = nidx.astype(jnp.int32)
    valid = (vals < 2.0e9).astype(jnp.int32)
    cnt = jnp.sum(valid, axis=-1)
    nidx = jnp.where(valid > 0, nidx, 0)
    return nidx, valid, cnt


def kernel(x, pos, batch, params):
    del batch
    x0 = x.reshape(B, N, F_IN)
    p0 = pos.reshape(B, N, 3)
    p0x, p0y, p0z = p0[..., 0], p0[..., 1], p0[..., 2]

    _, s1x, s1y, s1z = _fps(p0x, p0y, p0z, N, S1)
    q1x, q1y, q1z = (_plane_pad(s, S1P) for s in (s1x, s1y, s1z))
    keys1 = _select(q1x, q1y, q1z, p0x, p0y, p0z, RADII[0] ** 2,
                    MAX_NEIGH, qblk=208)
    table1 = jnp.concatenate(
        [x0, p0, jnp.zeros((B, N, 10), jnp.float32)], axis=-1).reshape(B * N, 16)
    nidx1, val1, cnt1 = _neighbors(keys1)
    gid1 = (jnp.arange(B, dtype=jnp.int32)[:, None, None] * N + nidx1).reshape(-1)
    G1 = table1[gid1]
    qpos1 = jnp.stack([q1x, q1y, q1z], axis=-1).reshape(B * S1P, 3)
    x1 = _sa_mlp(G1, qpos1, cnt1.reshape(-1, 1), params['sa1'], MAX_NEIGH,
                 cin_split=3, qblk=104, vflags=val1.reshape(-1, 1))
    x1 = x1.reshape(B, S1P, 128)

    c1x, c1y, c1z = (_plane_pad(s, NP1) for s in (s1x, s1y, s1z))
    c1x, c1y, c1z = _tie(cnt1, c1x, c1y, c1z)
    _, s2x, s2y, s2z = _fps(c1x, c1y, c1z, S1, S2)
    q2x, q2y, q2z = (_plane_pad(s, S2P) for s in (s2x, s2y, s2z))
    keys2 = _select(q2x, q2y, q2z, c1x, c1y, c1z, RADII[1] ** 2,
                    MAX_NEIGH, qblk=256)
    x1w = jnp.pad(x1, ((0, 0), (0, NP1 - S1P), (0, 0)))
    table2 = jnp.concatenate(
        [x1w, jnp.stack([c1x, c1y, c1z], axis=-1),
         jnp.zeros((B, NP1, 13), jnp.float32)], axis=-1).reshape(B * NP1, 144)
    nidx2, val2, cnt2 = _neighbors(keys2)
    gid2 = (jnp.arange(B, dtype=jnp.int32)[:, None, None] * NP1 + nidx2).reshape(-1)
    G2 = table2[gid2]
    qpos2 = jnp.stack([q2x, q2y, q2z], axis=-1).reshape(B * S2P, 3)
    x2 = _sa_mlp(G2, qpos2, cnt2.reshape(-1, 1), params['sa2'], MAX_NEIGH,
                 cin_split=128, qblk=64, vflags=val2.reshape(-1, 1))
    x2 = x2.reshape(B, S2P, 256)

    x2 = _tie(cnt2, x2)
    pos2 = jnp.stack([q2x, q2y, q2z], axis=-1)
    f3 = _sa3_fp3(x2, pos2, params['sa3'], params['fp3'], S2)

    f2 = _interp_mlp((q1x, q1y, q1z), (q2x, q2y, q2z), f3, x1,
                     params['fp2'], c_src=256, qblk=208)

    f2w = jnp.pad(f2, ((0, 0), (0, NP1 - S1P), (0, 0)))
    f1 = _interp_mlp((p0x, p0y, p0z), (c1x, c1y, c1z), f2w, x0,
                     params['fp1'], c_src=128, qblk=512)

    out = _head(f1.reshape(B * N, 128), params['out'])
    return out.reshape(B * N, NUM_CLASSES)
